# baseline (speedup 1.0000x reference)
import functools

import jax
import jax.numpy as jnp
from jax import lax
from jax.experimental import pallas as pl
from jax.experimental.pallas import tpu as pltpu
from jax.experimental.pallas import tpu_sc as plsc

N = 10000
NPAD = 10240
NE = 160000
D = 256
ED = 7
EDP = 8

NC, NS, L = 2, 16, 16
NW = NC * NS
NV = D // L
NPASS = 2
RPP = NPAD // NW // NPASS

BLK = 800
VPB = BLK // L
NBLK = NE // BLK

K = 64
NB = 2
THRESH = 512
WLSZ = THRESH + BLK + 80


def _rsqrt_vec(d):
    i = lax.bitcast_convert_type(d, jnp.int32)
    i = jnp.int32(0x5F3759DF) - (i >> 1)
    y = lax.bitcast_convert_type(i, jnp.float32)
    for _ in range(3):
        y = y * (1.5 - 0.5 * d * y * y)
    return y


def _sc_body(row_hbm, col_hbm, h_hbm, e_hbm, out_hbm, dinv_hbm,
             rb0, rb1, cb0, cb1, ones_blk, dinv, wl_row, wl_eid, wl_col,
             nrm_buf, acc, h_b0, h_b1, e_b0, e_b1,
             cnt_sh, sh0, sh1, se0, se1, sr0, sr1, sc0, sc1):
    c = lax.axis_index("c")
    s = lax.axis_index("s")
    wid = s * NC + c
    zf = jnp.zeros((L,), jnp.float32)
    zi = jnp.zeros((L,), jnp.int32)
    iota = lax.iota(jnp.int32, L)
    h_bufs = (h_b0, h_b1)
    e_bufs = (e_b0, e_b1)
    sems_h = (sh0, sh1)
    sems_e = (se0, se1)
    rbufs = (rb0, rb1)
    cbufs = (cb0, cb1)
    sems_r = (sr0, sr1)
    sems_c = (sc0, sc1)

    def _fill_ones(i, carry):
        ones_blk[pl.ds(i * L, L)] = zf + 1.0
        return carry
    lax.fori_loop(0, BLK // L, _fill_ones, 0)

    def _zero_dinv(i, carry):
        dinv[pl.ds(i * L, L)] = zf
        return carry
    lax.fori_loop(0, (NPAD + L) // L, _zero_dinv, 0)

    @pl.when(s == 0)
    def _():
        pltpu.sync_copy(dinv.at[pl.ds(0, NPAD)], cnt_sh)
    plsc.subcore_barrier()

    def _count_blk(b, carry):
        idx = b * NS + s
        @pl.when(idx < NBLK)
        def _():
            pltpu.sync_copy(row_hbm.at[pl.ds(idx * BLK, BLK)], rb0)
            pltpu.sync_copy(ones_blk, cnt_sh.at[rb0], add=True)
        return carry
    lax.fori_loop(0, (NBLK + NS - 1) // NS, _count_blk, 0)
    plsc.subcore_barrier()

    pltpu.sync_copy(cnt_sh, dinv.at[pl.ds(0, NPAD)])

    def _mk_dinv(i, carry):
        d = dinv[pl.ds(i * L, L)] + 1.0
        dinv[pl.ds(i * L, L)] = _rsqrt_vec(d)
        return carry
    lax.fori_loop(0, NPAD // L, _mk_dinv, 0)

    @pl.when(wid == 0)
    def _():
        pltpu.sync_copy(dinv.at[pl.ds(0, NPAD)], dinv_hbm)

    def _start(k, b):
        pltpu.make_async_copy(
            h_hbm.at[wl_row.at[pl.ds(k * K, K)]], h_bufs[b], sems_h[b]).start()
        pltpu.make_async_copy(
            e_hbm.at[wl_eid.at[pl.ds(k * K, K)]], e_bufs[b], sems_e[b]).start()

    def _make_pass(p):
        vbase = wid * NPASS + p
        rbase = vbase * RPP

        def _zero_acc(j, carry):
            acc[pl.ds(j * L, L)] = zf
            return carry
        lax.fori_loop(0, RPP * D // L, _zero_acc, 0)

        def _finish(k, b, lim):
            pltpu.make_async_copy(
                h_hbm.at[wl_row.at[pl.ds(k * K, K)]], h_bufs[b],
                sems_h[b]).wait()
            pltpu.make_async_copy(
                e_hbm.at[wl_eid.at[pl.ds(k * K, K)]], e_bufs[b],
                sems_e[b]).wait()
            nv = lim - k * K
            for t in range(K // L):
                rv = wl_row[pl.ds(k * K + t * L, L)]
                cv = wl_col[pl.ds(k * K + t * L, L)]
                nrm = (plsc.load_gather(dinv, [rv]) *
                       plsc.load_gather(dinv, [cv]))
                valid = (iota + t * L) < nv
                nrm_buf[pl.ds(t * L, L)] = jnp.where(valid, nrm, 0.0)
            h_buf = h_bufs[b]
            e_buf = e_bufs[b]

            def _edge(j, carry):
                nrm = nrm_buf[pl.ds(j, L)][0]
                cl = wl_col[pl.ds(k * K + j, L)][0] - rbase
                base = cl << 8
                for i in range(NV):
                    hv = h_buf[j, pl.ds(i * L, L)]
                    ev = e_buf[j, pl.ds(i * L, L)]
                    plsc.addupdate(acc.at[pl.ds(base + i * L, L)],
                                   jnp.maximum(hv + ev, 0.0) * nrm)
                return carry
            lax.fori_loop(0, K, _edge, 0, unroll=2)

        def _drain(nch, lim):
            @pl.when(0 < nch)
            def _():
                _start(0, 0)
            ngroups = (nch + 1) >> 1

            def _grp(g, carry):
                k0 = g * NB
                for b in range(NB):
                    k = k0 + b
                    nxt = k + NB - 1

                    @pl.when(nxt < nch)
                    def _(nxt=nxt, b=b):
                        _start(nxt, (b + NB - 1) % NB)

                    @pl.when(k < nch)
                    def _(k=k, b=b):
                        _finish(k, b, lim)
                return carry
            lax.fori_loop(0, ngroups, _grp, 0)

        def _scan_vec(bufs, bbase, v, off):
            rv = bufs[0][pl.ds(v * L, L)]
            cv = bufs[1][pl.ds(v * L, L)]
            owner = ((cv >> 5) * 6554) >> 15
            m = owner == vbase
            eid = (bbase + v * L) + iota
            plsc.store_compressed(wl_row.at[pl.ds(off, L)], rv, mask=m)
            plsc.store_compressed(wl_eid.at[pl.ds(off, L)], eid, mask=m)
            plsc.store_compressed(wl_col.at[pl.ds(off, L)], cv, mask=m)
            return off + plsc.all_reduce_population_count(m)[0]

        def _start_blk(b, buf):
            pltpu.make_async_copy(
                row_hbm.at[pl.ds(b * BLK, BLK)], rbufs[buf],
                sems_r[buf]).start()
            pltpu.make_async_copy(
                col_hbm.at[pl.ds(b * BLK, BLK)], cbufs[buf],
                sems_c[buf]).start()

        def _scan_blk(buf, b, off):
            pltpu.make_async_copy(
                row_hbm.at[pl.ds(b * BLK, BLK)], rbufs[buf],
                sems_r[buf]).wait()
            pltpu.make_async_copy(
                col_hbm.at[pl.ds(b * BLK, BLK)], cbufs[buf],
                sems_c[buf]).wait()
            return lax.fori_loop(
                0, VPB,
                functools.partial(_scan_vec, (rbufs[buf], cbufs[buf]),
                                  b * BLK),
                off)

        def _pad_tail(off):
            for t in range(K // L):
                wl_row[pl.ds(off + t * L, L)] = zi
                wl_eid[pl.ds(off + t * L, L)] = zi
                wl_col[pl.ds(off + t * L, L)] = zi + rbase
            return off

        _start_blk(0, 0)

        def _body(g, off):
            for sub in range(NB):
                b = g * NB + sub

                def _real(off, b=b, sub=sub):
                    nxt = b + 1

                    @pl.when(nxt < NBLK)
                    def _():
                        _start_blk(nxt, (sub + 1) % NB)

                    is_last = b == NBLK
                    off = lax.cond(
                        is_last, _pad_tail,
                        functools.partial(_scan_blk, sub, b), off)
                    do = (off >= THRESH) | is_last
                    nch = lax.select(is_last, (off + (K - 1)) >> 6, off >> 6)
                    lim = nch << 6

                    def _do_drain(off):
                        _drain(nch, jnp.minimum(lim, off))
                        base = (off >> 6) << 6
                        for t in range(K // L):
                            lr = wl_row[pl.ds(base + t * L, L)]
                            le = wl_eid[pl.ds(base + t * L, L)]
                            lc = wl_col[pl.ds(base + t * L, L)]
                            wl_row[pl.ds(t * L, L)] = lr
                            wl_eid[pl.ds(t * L, L)] = le
                            wl_col[pl.ds(t * L, L)] = lc
                        return off - base

                    return lax.cond(do, _do_drain, lambda o: o, off)

                off = lax.cond(b <= NBLK, _real, lambda o: o, off)
            return off
        lax.fori_loop(0, (NBLK + 2 + NB - 1) // NB, _body, jnp.int32(0))

        pltpu.sync_copy(acc, out_hbm.at[pl.ds(rbase * D, RPP * D)])

    for p in range(NPASS):
        _make_pass(p)


_sc_main = pl.kernel(
    _sc_body,
    out_type=(jax.ShapeDtypeStruct((NPAD * D,), jnp.float32),
              jax.ShapeDtypeStruct((NPAD,), jnp.float32)),
    mesh=plsc.VectorSubcoreMesh(core_axis_name="c", subcore_axis_name="s"),
    compiler_params=pltpu.CompilerParams(needs_layout_passes=False),
    scratch_types=[
        pltpu.VMEM((BLK,), jnp.int32),
        pltpu.VMEM((BLK,), jnp.int32),
        pltpu.VMEM((BLK,), jnp.int32),
        pltpu.VMEM((BLK,), jnp.int32),
        pltpu.VMEM((BLK,), jnp.float32),
        pltpu.VMEM((NPAD + L,), jnp.float32),
        pltpu.VMEM((WLSZ,), jnp.int32),
        pltpu.VMEM((WLSZ,), jnp.int32),
        pltpu.VMEM((WLSZ,), jnp.int32),
        pltpu.VMEM((K + L,), jnp.float32),
        pltpu.VMEM((RPP * D,), jnp.float32),
        pltpu.VMEM((K, D), jnp.float32),
        pltpu.VMEM((K, D), jnp.float32),
        pltpu.VMEM((K, D), jnp.float32),
        pltpu.VMEM((K, D), jnp.float32),
        pltpu.VMEM_SHARED((NPAD,), jnp.float32),
        pltpu.SemaphoreType.DMA,
        pltpu.SemaphoreType.DMA,
        pltpu.SemaphoreType.DMA,
        pltpu.SemaphoreType.DMA,
        pltpu.SemaphoreType.DMA,
        pltpu.SemaphoreType.DMA,
        pltpu.SemaphoreType.DMA,
        pltpu.SemaphoreType.DMA,
    ],
)


def _h_body(x_ref, w_ref, b_ref, o_ref):
    o_ref[...] = lax.dot_general(
        x_ref[...], w_ref[...], (((1,), (1,)), ((), ())),
        preferred_element_type=jnp.float32) + b_ref[...]


_h_call = pl.pallas_call(
    _h_body,
    grid=(5,),
    in_specs=[
        pl.BlockSpec((N // 5, D), lambda i: (i, 0)),
        pl.BlockSpec((D, D), lambda i: (0, 0)),
        pl.BlockSpec((1, D), lambda i: (0, 0)),
    ],
    out_specs=pl.BlockSpec((N // 5, D), lambda i: (i, 0)),
    out_shape=jax.ShapeDtypeStruct((N, D), jnp.float32),
)


def _e_body(a_ref, w_ref, b_ref, o_ref):
    o_ref[...] = lax.dot_general(
        a_ref[...], w_ref[...], (((1,), (0,)), ((), ())),
        preferred_element_type=jnp.float32) + b_ref[...]


_e_call = pl.pallas_call(
    _e_body,
    grid=(80,),
    in_specs=[
        pl.BlockSpec((2000, EDP), lambda i: (i, 0)),
        pl.BlockSpec((EDP, D), lambda i: (0, 0)),
        pl.BlockSpec((1, D), lambda i: (0, 0)),
    ],
    out_specs=pl.BlockSpec((2000, D), lambda i: (i, 0)),
    out_shape=jax.ShapeDtypeStruct((NE, D), jnp.float32),
)


def _fin_body(a_ref, h_ref, d_ref, r_ref, o_ref):
    dd = d_ref[...] * d_ref[...]
    o_ref[...] = a_ref[...] + jnp.maximum(h_ref[...] + r_ref[...], 0.0) * dd


_fin_call = pl.pallas_call(
    _fin_body,
    grid=(5,),
    in_specs=[
        pl.BlockSpec((N // 5, D), lambda i: (i, 0)),
        pl.BlockSpec((N // 5, D), lambda i: (i, 0)),
        pl.BlockSpec((N // 5, 1), lambda i: (i, 0)),
        pl.BlockSpec((1, D), lambda i: (0, 0)),
    ],
    out_specs=pl.BlockSpec((N // 5, D), lambda i: (i, 0)),
    out_shape=jax.ShapeDtypeStruct((N, D), jnp.float32),
)


@jax.jit
def kernel(x, edge_index, edge_attr, W_lin, b_lin, W_edge, b_edge, root_emb):
    ei = edge_index.astype(jnp.int32)
    row = ei[0]
    col = ei[1]
    attr_pad = jnp.pad(edge_attr, ((0, 0), (0, EDP - ED)))
    w_e = jnp.pad(W_edge.T, ((0, EDP - ED), (0, 0)))
    h = _h_call(x, W_lin, b_lin.reshape(1, D))
    e = _e_call(attr_pad, w_e, b_edge.reshape(1, D))
    aggr, dinv = _sc_main(row, col, h, e)
    aggr = aggr.reshape(NPAD, D)
    out = _fin_call(aggr[:N], h, dinv[:N].reshape(N, 1),
                    root_emb.reshape(1, D))
    return out


# speedup vs baseline: 1.3684x; 1.3684x over previous
import functools

import jax
import jax.numpy as jnp
from jax import lax
from jax.experimental import pallas as pl
from jax.experimental.pallas import tpu as pltpu
from jax.experimental.pallas import tpu_sc as plsc

N = 10000
NPAD = 10240
NE = 160000
D = 256
ED = 7
EDP = 8

NC, NS, L = 2, 16, 16
NW = NC * NS
NV = D // L
NPASS = 2
RPP = NPAD // NW // NPASS

BLK = 800
VPB = BLK // L
NBLK = NE // BLK

K = 64
NB = 2
THRESH = 512
WLSZ = THRESH + BLK + 80


def _rsqrt_vec(d):
    i = lax.bitcast_convert_type(d, jnp.int32)
    i = jnp.int32(0x5F3759DF) - (i >> 1)
    y = lax.bitcast_convert_type(i, jnp.float32)
    for _ in range(3):
        y = y * (1.5 - 0.5 * d * y * y)
    return y


def _sc_body(row_hbm, col_hbm, h_hbm, e_hbm, out_hbm, dinv_hbm,
             rb0, rb1, cb0, cb1, ones_blk, dinv, wl_row, wl_eid, wl_col,
             nrm_buf, acc, h_b0, h_b1, e_b0, e_b1,
             cnt_sh, sh0, sh1, se0, se1, sr0, sr1, sc0, sc1):
    c = lax.axis_index("c")
    s = lax.axis_index("s")
    wid = s * NC + c
    zf = jnp.zeros((L,), jnp.float32)
    zi = jnp.zeros((L,), jnp.int32)
    iota = lax.iota(jnp.int32, L)
    h_bufs = (h_b0, h_b1)
    e_bufs = (e_b0, e_b1)
    sems_h = (sh0, sh1)
    sems_e = (se0, se1)
    rbufs = (rb0, rb1)
    cbufs = (cb0, cb1)
    sems_r = (sr0, sr1)
    sems_c = (sc0, sc1)

    def _fill_ones(i, carry):
        ones_blk[pl.ds(i * L, L)] = zf + 1.0
        return carry
    lax.fori_loop(0, BLK // L, _fill_ones, 0)

    def _zero_dinv(i, carry):
        dinv[pl.ds(i * L, L)] = zf
        return carry
    lax.fori_loop(0, (NPAD + L) // L, _zero_dinv, 0)

    @pl.when(s == 0)
    def _():
        pltpu.sync_copy(dinv.at[pl.ds(0, NPAD)], cnt_sh)
    plsc.subcore_barrier()

    def _count_blk(b, carry):
        idx = b * NS + s
        @pl.when(idx < NBLK)
        def _():
            pltpu.sync_copy(row_hbm.at[pl.ds(idx * BLK, BLK)], rb0)
            pltpu.sync_copy(ones_blk, cnt_sh.at[rb0], add=True)
        return carry
    lax.fori_loop(0, (NBLK + NS - 1) // NS, _count_blk, 0)
    plsc.subcore_barrier()

    pltpu.sync_copy(cnt_sh, dinv.at[pl.ds(0, NPAD)])

    def _mk_dinv(i, carry):
        d = dinv[pl.ds(i * L, L)] + 1.0
        dinv[pl.ds(i * L, L)] = _rsqrt_vec(d)
        return carry
    lax.fori_loop(0, NPAD // L, _mk_dinv, 0)

    @pl.when(wid == 0)
    def _():
        pltpu.sync_copy(dinv.at[pl.ds(0, NPAD)], dinv_hbm)

    def _start(k, b):
        pltpu.make_async_copy(
            h_hbm.at[wl_row.at[pl.ds(k * K, K)]], h_bufs[b], sems_h[b]).start()
        pltpu.make_async_copy(
            e_hbm.at[wl_eid.at[pl.ds(k * K, K)]], e_bufs[b], sems_e[b]).start()

    def _make_pass(p):
        vbase = wid * NPASS + p
        rbase = vbase * RPP

        def _zero_acc(j, carry):
            acc[pl.ds(j * L, L)] = zf
            return carry
        lax.fori_loop(0, RPP * D // L, _zero_acc, 0)

        def _finish(k, b, lim):
            pltpu.make_async_copy(
                h_hbm.at[wl_row.at[pl.ds(k * K, K)]], h_bufs[b],
                sems_h[b]).wait()
            pltpu.make_async_copy(
                e_hbm.at[wl_eid.at[pl.ds(k * K, K)]], e_bufs[b],
                sems_e[b]).wait()
            nv = lim - k * K
            for t in range(K // L):
                rv = wl_row[pl.ds(k * K + t * L, L)]
                cv = wl_col[pl.ds(k * K + t * L, L)]
                nrm = (plsc.load_gather(dinv, [rv]) *
                       plsc.load_gather(dinv, [cv]))
                valid = (iota + t * L) < nv
                nrm_buf[pl.ds(t * L, L)] = jnp.where(valid, nrm, 0.0)
            h_buf = h_bufs[b]
            e_buf = e_bufs[b]

            def _edge(j, carry):
                nrm = nrm_buf[pl.ds(j, L)][0]
                cl = wl_col[pl.ds(k * K + j, L)][0] - rbase
                base = cl << 8
                for g in range(NV // 4):
                    hv = [h_buf[j, pl.ds((g * 4 + i) * L, L)]
                          for i in range(4)]
                    ev = [e_buf[j, pl.ds((g * 4 + i) * L, L)]
                          for i in range(4)]
                    sv = [hv[i] + ev[i] for i in range(4)]
                    mv = [jnp.maximum(sv[i], 0.0) * nrm for i in range(4)]
                    for i in range(4):
                        plsc.addupdate(
                            acc.at[pl.ds(base + (g * 4 + i) * L, L)], mv[i])
                return carry
            lax.fori_loop(0, K, _edge, 0, unroll=2)

        def _drain(nch, lim):
            @pl.when(0 < nch)
            def _():
                _start(0, 0)
            ngroups = (nch + 1) >> 1

            def _grp(g, carry):
                k0 = g * NB
                for b in range(NB):
                    k = k0 + b
                    nxt = k + NB - 1

                    @pl.when(nxt < nch)
                    def _(nxt=nxt, b=b):
                        _start(nxt, (b + NB - 1) % NB)

                    @pl.when(k < nch)
                    def _(k=k, b=b):
                        _finish(k, b, lim)
                return carry
            lax.fori_loop(0, ngroups, _grp, 0)

        def _scan_vec(bufs, bbase, v, off):
            rv = bufs[0][pl.ds(v * L, L)]
            cv = bufs[1][pl.ds(v * L, L)]
            owner = ((cv >> 5) * 6554) >> 15
            m = owner == vbase
            eid = (bbase + v * L) + iota
            plsc.store_compressed(wl_row.at[pl.ds(off, L)], rv, mask=m)
            plsc.store_compressed(wl_eid.at[pl.ds(off, L)], eid, mask=m)
            plsc.store_compressed(wl_col.at[pl.ds(off, L)], cv, mask=m)
            return off + plsc.all_reduce_population_count(m)[0]

        def _start_blk(b, buf):
            pltpu.make_async_copy(
                row_hbm.at[pl.ds(b * BLK, BLK)], rbufs[buf],
                sems_r[buf]).start()
            pltpu.make_async_copy(
                col_hbm.at[pl.ds(b * BLK, BLK)], cbufs[buf],
                sems_c[buf]).start()

        def _scan_blk(buf, b, off):
            pltpu.make_async_copy(
                row_hbm.at[pl.ds(b * BLK, BLK)], rbufs[buf],
                sems_r[buf]).wait()
            pltpu.make_async_copy(
                col_hbm.at[pl.ds(b * BLK, BLK)], cbufs[buf],
                sems_c[buf]).wait()
            return lax.fori_loop(
                0, VPB,
                functools.partial(_scan_vec, (rbufs[buf], cbufs[buf]),
                                  b * BLK),
                off)

        def _pad_tail(off):
            for t in range(K // L):
                wl_row[pl.ds(off + t * L, L)] = zi
                wl_eid[pl.ds(off + t * L, L)] = zi
                wl_col[pl.ds(off + t * L, L)] = zi + rbase
            return off

        _start_blk(0, 0)

        def _body(g, off):
            for sub in range(NB):
                b = g * NB + sub

                def _real(off, b=b, sub=sub):
                    nxt = b + 1

                    @pl.when(nxt < NBLK)
                    def _():
                        _start_blk(nxt, (sub + 1) % NB)

                    is_last = b == NBLK
                    off = lax.cond(
                        is_last, _pad_tail,
                        functools.partial(_scan_blk, sub, b), off)
                    do = (off >= THRESH) | is_last
                    nch = lax.select(is_last, (off + (K - 1)) >> 6, off >> 6)
                    lim = nch << 6

                    def _do_drain(off):
                        _drain(nch, jnp.minimum(lim, off))
                        base = (off >> 6) << 6
                        for t in range(K // L):
                            lr = wl_row[pl.ds(base + t * L, L)]
                            le = wl_eid[pl.ds(base + t * L, L)]
                            lc = wl_col[pl.ds(base + t * L, L)]
                            wl_row[pl.ds(t * L, L)] = lr
                            wl_eid[pl.ds(t * L, L)] = le
                            wl_col[pl.ds(t * L, L)] = lc
                        return off - base

                    return lax.cond(do, _do_drain, lambda o: o, off)

                off = lax.cond(b <= NBLK, _real, lambda o: o, off)
            return off
        lax.fori_loop(0, (NBLK + 2 + NB - 1) // NB, _body, jnp.int32(0))

        pltpu.sync_copy(acc, out_hbm.at[pl.ds(rbase * D, RPP * D)])

    for p in range(NPASS):
        _make_pass(p)


_sc_main = pl.kernel(
    _sc_body,
    out_type=(jax.ShapeDtypeStruct((NPAD * D,), jnp.float32),
              jax.ShapeDtypeStruct((NPAD,), jnp.float32)),
    mesh=plsc.VectorSubcoreMesh(core_axis_name="c", subcore_axis_name="s"),
    compiler_params=pltpu.CompilerParams(needs_layout_passes=False),
    scratch_types=[
        pltpu.VMEM((BLK,), jnp.int32),
        pltpu.VMEM((BLK,), jnp.int32),
        pltpu.VMEM((BLK,), jnp.int32),
        pltpu.VMEM((BLK,), jnp.int32),
        pltpu.VMEM((BLK,), jnp.float32),
        pltpu.VMEM((NPAD + L,), jnp.float32),
        pltpu.VMEM((WLSZ,), jnp.int32),
        pltpu.VMEM((WLSZ,), jnp.int32),
        pltpu.VMEM((WLSZ,), jnp.int32),
        pltpu.VMEM((K + L,), jnp.float32),
        pltpu.VMEM((RPP * D,), jnp.float32),
        pltpu.VMEM((K, D), jnp.float32),
        pltpu.VMEM((K, D), jnp.float32),
        pltpu.VMEM((K, D), jnp.float32),
        pltpu.VMEM((K, D), jnp.float32),
        pltpu.VMEM_SHARED((NPAD,), jnp.float32),
        pltpu.SemaphoreType.DMA,
        pltpu.SemaphoreType.DMA,
        pltpu.SemaphoreType.DMA,
        pltpu.SemaphoreType.DMA,
        pltpu.SemaphoreType.DMA,
        pltpu.SemaphoreType.DMA,
        pltpu.SemaphoreType.DMA,
        pltpu.SemaphoreType.DMA,
    ],
)


def _h_body(x_ref, w_ref, b_ref, o_ref):
    o_ref[...] = lax.dot_general(
        x_ref[...], w_ref[...], (((1,), (1,)), ((), ())),
        preferred_element_type=jnp.float32) + b_ref[...]


_h_call = pl.pallas_call(
    _h_body,
    grid=(5,),
    in_specs=[
        pl.BlockSpec((N // 5, D), lambda i: (i, 0)),
        pl.BlockSpec((D, D), lambda i: (0, 0)),
        pl.BlockSpec((1, D), lambda i: (0, 0)),
    ],
    out_specs=pl.BlockSpec((N // 5, D), lambda i: (i, 0)),
    out_shape=jax.ShapeDtypeStruct((N, D), jnp.float32),
)


def _e_body(a_ref, w_ref, b_ref, o_ref):
    o_ref[...] = lax.dot_general(
        a_ref[...], w_ref[...], (((1,), (0,)), ((), ())),
        preferred_element_type=jnp.float32) + b_ref[...]


_e_call = pl.pallas_call(
    _e_body,
    grid=(80,),
    in_specs=[
        pl.BlockSpec((2000, EDP), lambda i: (i, 0)),
        pl.BlockSpec((EDP, D), lambda i: (0, 0)),
        pl.BlockSpec((1, D), lambda i: (0, 0)),
    ],
    out_specs=pl.BlockSpec((2000, D), lambda i: (i, 0)),
    out_shape=jax.ShapeDtypeStruct((NE, D), jnp.float32),
)


def _fin_body(a_ref, h_ref, d_ref, r_ref, o_ref):
    dd = d_ref[...] * d_ref[...]
    o_ref[...] = a_ref[...] + jnp.maximum(h_ref[...] + r_ref[...], 0.0) * dd


_fin_call = pl.pallas_call(
    _fin_body,
    grid=(5,),
    in_specs=[
        pl.BlockSpec((N // 5, D), lambda i: (i, 0)),
        pl.BlockSpec((N // 5, D), lambda i: (i, 0)),
        pl.BlockSpec((N // 5, 1), lambda i: (i, 0)),
        pl.BlockSpec((1, D), lambda i: (0, 0)),
    ],
    out_specs=pl.BlockSpec((N // 5, D), lambda i: (i, 0)),
    out_shape=jax.ShapeDtypeStruct((N, D), jnp.float32),
)


@jax.jit
def kernel(x, edge_index, edge_attr, W_lin, b_lin, W_edge, b_edge, root_emb):
    ei = edge_index.astype(jnp.int32)
    row = ei[0]
    col = ei[1]
    attr_pad = jnp.pad(edge_attr, ((0, 0), (0, EDP - ED)))
    w_e = jnp.pad(W_edge.T, ((0, EDP - ED), (0, 0)))
    h = _h_call(x, W_lin, b_lin.reshape(1, D))
    e = _e_call(attr_pad, w_e, b_edge.reshape(1, D))
    aggr, dinv = _sc_main(row, col, h, e)
    aggr = aggr.reshape(NPAD, D)
    out = _fin_call(aggr[:N], h, dinv[:N].reshape(N, 1),
                    root_emb.reshape(1, D))
    return out


# speedup vs baseline: 1.4276x; 1.0432x over previous
import functools

import jax
import jax.numpy as jnp
from jax import lax
from jax.experimental import pallas as pl
from jax.experimental.pallas import tpu as pltpu
from jax.experimental.pallas import tpu_sc as plsc

N = 10000
NPAD = 10240
NE = 160000
D = 256
ED = 7
EDP = 8

NC, NS, L = 2, 16, 16
NW = NC * NS
NV = D // L
NPASS = 2
RPP = NPAD // NW // NPASS

BLK = 800
VPB = BLK // L
NBLK = NE // BLK

K = 64
NB = 2
THRESH = 512
WLSZ = THRESH + BLK + 80


def _rsqrt_vec(d):
    i = lax.bitcast_convert_type(d, jnp.int32)
    i = jnp.int32(0x5F3759DF) - (i >> 1)
    y = lax.bitcast_convert_type(i, jnp.float32)
    for _ in range(3):
        y = y * (1.5 - 0.5 * d * y * y)
    return y


def _sc_body(row_hbm, col_hbm, h_hbm, e_hbm, out_hbm, dinv_hbm,
             rb0, rb1, cb0, cb1, ones_blk, dinv, wl_row, wl_eid, wl_col,
             wlb_row, wlb_eid, wlb_col,
             nrm_buf, acc, h_b0, h_b1, e_b0, e_b1,
             cnt_sh, sh0, sh1, se0, se1, sr0, sr1, sc0, sc1):
    c = lax.axis_index("c")
    s = lax.axis_index("s")
    wid = s * NC + c
    zf = jnp.zeros((L,), jnp.float32)
    zi = jnp.zeros((L,), jnp.int32)
    iota = lax.iota(jnp.int32, L)
    h_bufs = (h_b0, h_b1)
    e_bufs = (e_b0, e_b1)
    sems_h = (sh0, sh1)
    sems_e = (se0, se1)
    rbufs = (rb0, rb1)
    cbufs = (cb0, cb1)
    sems_r = (sr0, sr1)
    sems_c = (sc0, sc1)

    def _fill_ones(i, carry):
        ones_blk[pl.ds(i * L, L)] = zf + 1.0
        return carry
    lax.fori_loop(0, BLK // L, _fill_ones, 0)

    def _zero_dinv(i, carry):
        dinv[pl.ds(i * L, L)] = zf
        return carry
    lax.fori_loop(0, (NPAD + L) // L, _zero_dinv, 0)

    @pl.when(s == 0)
    def _():
        pltpu.sync_copy(dinv.at[pl.ds(0, NPAD)], cnt_sh)
    plsc.subcore_barrier()

    def _count_blk(b, carry):
        idx = b * NS + s
        @pl.when(idx < NBLK)
        def _():
            pltpu.sync_copy(row_hbm.at[pl.ds(idx * BLK, BLK)], rb0)
            pltpu.sync_copy(ones_blk, cnt_sh.at[rb0], add=True)
        return carry
    lax.fori_loop(0, (NBLK + NS - 1) // NS, _count_blk, 0)
    plsc.subcore_barrier()

    pltpu.sync_copy(cnt_sh, dinv.at[pl.ds(0, NPAD)])

    def _mk_dinv(i, carry):
        d = dinv[pl.ds(i * L, L)] + 1.0
        dinv[pl.ds(i * L, L)] = _rsqrt_vec(d)
        return carry
    lax.fori_loop(0, NPAD // L, _mk_dinv, 0)

    @pl.when(wid == 0)
    def _():
        pltpu.sync_copy(dinv.at[pl.ds(0, NPAD)], dinv_hbm)

    def _start(k, b):
        pltpu.make_async_copy(
            h_hbm.at[wl_row.at[pl.ds(k * K, K)]], h_bufs[b], sems_h[b]).start()
        pltpu.make_async_copy(
            e_hbm.at[wl_eid.at[pl.ds(k * K, K)]], e_bufs[b], sems_e[b]).start()

    def _make_pass(p):
        vbase = wid * NPASS + p
        rbase = vbase * RPP

        def _zero_acc(j, carry):
            acc[pl.ds(j * L, L)] = zf
            return carry
        lax.fori_loop(0, RPP * D // L, _zero_acc, 0)

        def _finish(k, b, lim):
            pltpu.make_async_copy(
                h_hbm.at[wl_row.at[pl.ds(k * K, K)]], h_bufs[b],
                sems_h[b]).wait()
            pltpu.make_async_copy(
                e_hbm.at[wl_eid.at[pl.ds(k * K, K)]], e_bufs[b],
                sems_e[b]).wait()
            nv = lim - k * K
            for t in range(K // L):
                rv = wl_row[pl.ds(k * K + t * L, L)]
                cv = wl_col[pl.ds(k * K + t * L, L)]
                nrm = (plsc.load_gather(dinv, [rv]) *
                       plsc.load_gather(dinv, [cv]))
                valid = (iota + t * L) < nv
                nrm_buf[pl.ds(t * L, L)] = jnp.where(valid, nrm, 0.0)
            h_buf = h_bufs[b]
            e_buf = e_bufs[b]

            def _edge(j, carry):
                nrm = nrm_buf[pl.ds(j, L)][0]
                cl = wl_col[pl.ds(k * K + j, L)][0] - rbase
                base = cl << 8
                for g in range(NV // 4):
                    hv = [h_buf[j, pl.ds((g * 4 + i) * L, L)]
                          for i in range(4)]
                    ev = [e_buf[j, pl.ds((g * 4 + i) * L, L)]
                          for i in range(4)]
                    sv = [hv[i] + ev[i] for i in range(4)]
                    mv = [jnp.maximum(sv[i], 0.0) * nrm for i in range(4)]
                    for i in range(4):
                        plsc.addupdate(
                            acc.at[pl.ds(base + (g * 4 + i) * L, L)], mv[i])
                return carry
            lax.fori_loop(0, K, _edge, 0, unroll=2)

        def _drain(nch, lim):
            @pl.when(0 < nch)
            def _():
                _start(0, 0)
            ngroups = (nch + 1) >> 1

            def _grp(g, carry):
                k0 = g * NB
                for b in range(NB):
                    k = k0 + b
                    nxt = k + NB - 1

                    @pl.when(nxt < nch)
                    def _(nxt=nxt, b=b):
                        _start(nxt, (b + NB - 1) % NB)

                    @pl.when(k < nch)
                    def _(k=k, b=b):
                        _finish(k, b, lim)
                return carry
            lax.fori_loop(0, ngroups, _grp, 0)

        def _scan_pair(bufs, bbase, g, offs):
            offa, offb = offs
            va = 2 * g
            vb = 2 * g + 1
            ra = bufs[0][pl.ds(va * L, L)]
            ca = bufs[1][pl.ds(va * L, L)]
            rb = bufs[0][pl.ds(vb * L, L)]
            cb = bufs[1][pl.ds(vb * L, L)]
            ma = (((ca >> 5) * 6554) >> 15) == vbase
            mb = (((cb >> 5) * 6554) >> 15) == vbase
            ea = (bbase + va * L) + iota
            eb = (bbase + vb * L) + iota
            plsc.store_compressed(wl_row.at[pl.ds(offa, L)], ra, mask=ma)
            plsc.store_compressed(wlb_row.at[pl.ds(offb, L)], rb, mask=mb)
            plsc.store_compressed(wl_eid.at[pl.ds(offa, L)], ea, mask=ma)
            plsc.store_compressed(wlb_eid.at[pl.ds(offb, L)], eb, mask=mb)
            plsc.store_compressed(wl_col.at[pl.ds(offa, L)], ca, mask=ma)
            plsc.store_compressed(wlb_col.at[pl.ds(offb, L)], cb, mask=mb)
            pa = plsc.all_reduce_population_count(ma)[0]
            pb = plsc.all_reduce_population_count(mb)[0]
            return offa + pa, offb + pb

        def _start_blk(b, buf):
            pltpu.make_async_copy(
                row_hbm.at[pl.ds(b * BLK, BLK)], rbufs[buf],
                sems_r[buf]).start()
            pltpu.make_async_copy(
                col_hbm.at[pl.ds(b * BLK, BLK)], cbufs[buf],
                sems_c[buf]).start()

        def _scan_blk(buf, b, off):
            pltpu.make_async_copy(
                row_hbm.at[pl.ds(b * BLK, BLK)], rbufs[buf],
                sems_r[buf]).wait()
            pltpu.make_async_copy(
                col_hbm.at[pl.ds(b * BLK, BLK)], cbufs[buf],
                sems_c[buf]).wait()
            off, offb = lax.fori_loop(
                0, VPB // 2,
                functools.partial(_scan_pair, (rbufs[buf], cbufs[buf]),
                                  b * BLK),
                (off, jnp.int32(0)))

            def _merge(t, carry):
                mr = wlb_row[pl.ds(t * L, L)]
                me = wlb_eid[pl.ds(t * L, L)]
                mc = wlb_col[pl.ds(t * L, L)]
                wl_row[pl.ds(off + t * L, L)] = mr
                wl_eid[pl.ds(off + t * L, L)] = me
                wl_col[pl.ds(off + t * L, L)] = mc
                return carry
            lax.fori_loop(0, (offb + (L - 1)) >> 4, _merge, 0)
            return off + offb

        def _pad_tail(off):
            for t in range(K // L):
                wl_row[pl.ds(off + t * L, L)] = zi
                wl_eid[pl.ds(off + t * L, L)] = zi
                wl_col[pl.ds(off + t * L, L)] = zi + rbase
            return off

        _start_blk(0, 0)

        def _body(g, off):
            for sub in range(NB):
                b = g * NB + sub

                def _real(off, b=b, sub=sub):
                    nxt = b + 1

                    @pl.when(nxt < NBLK)
                    def _():
                        _start_blk(nxt, (sub + 1) % NB)

                    is_last = b == NBLK
                    off = lax.cond(
                        is_last, _pad_tail,
                        functools.partial(_scan_blk, sub, b), off)
                    do = (off >= THRESH) | is_last
                    nch = lax.select(is_last, (off + (K - 1)) >> 6, off >> 6)
                    lim = nch << 6

                    def _do_drain(off):
                        _drain(nch, jnp.minimum(lim, off))
                        base = (off >> 6) << 6
                        for t in range(K // L):
                            lr = wl_row[pl.ds(base + t * L, L)]
                            le = wl_eid[pl.ds(base + t * L, L)]
                            lc = wl_col[pl.ds(base + t * L, L)]
                            wl_row[pl.ds(t * L, L)] = lr
                            wl_eid[pl.ds(t * L, L)] = le
                            wl_col[pl.ds(t * L, L)] = lc
                        return off - base

                    return lax.cond(do, _do_drain, lambda o: o, off)

                off = lax.cond(b <= NBLK, _real, lambda o: o, off)
            return off
        lax.fori_loop(0, (NBLK + 2 + NB - 1) // NB, _body, jnp.int32(0))

        pltpu.sync_copy(acc, out_hbm.at[pl.ds(rbase * D, RPP * D)])

    for p in range(NPASS):
        _make_pass(p)


_sc_main = pl.kernel(
    _sc_body,
    out_type=(jax.ShapeDtypeStruct((NPAD * D,), jnp.float32),
              jax.ShapeDtypeStruct((NPAD,), jnp.float32)),
    mesh=plsc.VectorSubcoreMesh(core_axis_name="c", subcore_axis_name="s"),
    compiler_params=pltpu.CompilerParams(needs_layout_passes=False),
    scratch_types=[
        pltpu.VMEM((BLK,), jnp.int32),
        pltpu.VMEM((BLK,), jnp.int32),
        pltpu.VMEM((BLK,), jnp.int32),
        pltpu.VMEM((BLK,), jnp.int32),
        pltpu.VMEM((BLK,), jnp.float32),
        pltpu.VMEM((NPAD + L,), jnp.float32),
        pltpu.VMEM((WLSZ,), jnp.int32),
        pltpu.VMEM((WLSZ,), jnp.int32),
        pltpu.VMEM((WLSZ,), jnp.int32),
        pltpu.VMEM((BLK // 2 + L,), jnp.int32),
        pltpu.VMEM((BLK // 2 + L,), jnp.int32),
        pltpu.VMEM((BLK // 2 + L,), jnp.int32),
        pltpu.VMEM((K + L,), jnp.float32),
        pltpu.VMEM((RPP * D,), jnp.float32),
        pltpu.VMEM((K, D), jnp.float32),
        pltpu.VMEM((K, D), jnp.float32),
        pltpu.VMEM((K, D), jnp.float32),
        pltpu.VMEM((K, D), jnp.float32),
        pltpu.VMEM_SHARED((NPAD,), jnp.float32),
        pltpu.SemaphoreType.DMA,
        pltpu.SemaphoreType.DMA,
        pltpu.SemaphoreType.DMA,
        pltpu.SemaphoreType.DMA,
        pltpu.SemaphoreType.DMA,
        pltpu.SemaphoreType.DMA,
        pltpu.SemaphoreType.DMA,
        pltpu.SemaphoreType.DMA,
    ],
)


def _h_body(x_ref, w_ref, b_ref, o_ref):
    o_ref[...] = lax.dot_general(
        x_ref[...], w_ref[...], (((1,), (1,)), ((), ())),
        preferred_element_type=jnp.float32) + b_ref[...]


_h_call = pl.pallas_call(
    _h_body,
    grid=(5,),
    in_specs=[
        pl.BlockSpec((N // 5, D), lambda i: (i, 0)),
        pl.BlockSpec((D, D), lambda i: (0, 0)),
        pl.BlockSpec((1, D), lambda i: (0, 0)),
    ],
    out_specs=pl.BlockSpec((N // 5, D), lambda i: (i, 0)),
    out_shape=jax.ShapeDtypeStruct((N, D), jnp.float32),
)


def _e_body(a_ref, w_ref, b_ref, o_ref):
    o_ref[...] = lax.dot_general(
        a_ref[...], w_ref[...], (((1,), (0,)), ((), ())),
        preferred_element_type=jnp.float32) + b_ref[...]


_e_call = pl.pallas_call(
    _e_body,
    grid=(80,),
    in_specs=[
        pl.BlockSpec((2000, EDP), lambda i: (i, 0)),
        pl.BlockSpec((EDP, D), lambda i: (0, 0)),
        pl.BlockSpec((1, D), lambda i: (0, 0)),
    ],
    out_specs=pl.BlockSpec((2000, D), lambda i: (i, 0)),
    out_shape=jax.ShapeDtypeStruct((NE, D), jnp.float32),
)


def _fin_body(a_ref, h_ref, d_ref, r_ref, o_ref):
    dd = d_ref[...] * d_ref[...]
    o_ref[...] = a_ref[...] + jnp.maximum(h_ref[...] + r_ref[...], 0.0) * dd


_fin_call = pl.pallas_call(
    _fin_body,
    grid=(5,),
    in_specs=[
        pl.BlockSpec((N // 5, D), lambda i: (i, 0)),
        pl.BlockSpec((N // 5, D), lambda i: (i, 0)),
        pl.BlockSpec((N // 5, 1), lambda i: (i, 0)),
        pl.BlockSpec((1, D), lambda i: (0, 0)),
    ],
    out_specs=pl.BlockSpec((N // 5, D), lambda i: (i, 0)),
    out_shape=jax.ShapeDtypeStruct((N, D), jnp.float32),
)


@jax.jit
def kernel(x, edge_index, edge_attr, W_lin, b_lin, W_edge, b_edge, root_emb):
    ei = edge_index.astype(jnp.int32)
    row = ei[0]
    col = ei[1]
    attr_pad = jnp.pad(edge_attr, ((0, 0), (0, EDP - ED)))
    w_e = jnp.pad(W_edge.T, ((0, EDP - ED), (0, 0)))
    h = _h_call(x, W_lin, b_lin.reshape(1, D))
    e = _e_call(attr_pad, w_e, b_edge.reshape(1, D))
    aggr, dinv = _sc_main(row, col, h, e)
    aggr = aggr.reshape(NPAD, D)
    out = _fin_call(aggr[:N], h, dinv[:N].reshape(N, 1),
                    root_emb.reshape(1, D))
    return out


# speedup vs baseline: 1.4707x; 1.0302x over previous
import functools

import jax
import jax.numpy as jnp
from jax import lax
from jax.experimental import pallas as pl
from jax.experimental.pallas import tpu as pltpu
from jax.experimental.pallas import tpu_sc as plsc

N = 10000
NPAD = 10240
NE = 160000
D = 256
ED = 7
EDP = 8

NC, NS, L = 2, 16, 16
NW = NC * NS
NV = D // L
NPASS = 2
RPP = NPAD // NW // NPASS

BLK = 800
VPB = BLK // L
NBLK = NE // BLK

K = 64
NB = 2
THRESH = 512
WLSZ = THRESH + BLK + 80


def _rsqrt_vec(d):
    i = lax.bitcast_convert_type(d, jnp.int32)
    i = jnp.int32(0x5F3759DF) - (i >> 1)
    y = lax.bitcast_convert_type(i, jnp.float32)
    for _ in range(3):
        y = y * (1.5 - 0.5 * d * y * y)
    return y


def _cnt_body(row_hbm, dinv_hbm, idx_blk, ones_blk, dinv, cnt_sh):
    c = lax.axis_index("c")
    s = lax.axis_index("s")
    zf = jnp.zeros((L,), jnp.float32)

    def _fill_ones(i, carry):
        ones_blk[pl.ds(i * L, L)] = zf + 1.0
        return carry
    lax.fori_loop(0, BLK // L, _fill_ones, 0)

    def _zero_dinv(i, carry):
        dinv[pl.ds(i * L, L)] = zf
        return carry
    lax.fori_loop(0, NPAD // L, _zero_dinv, 0)

    @pl.when((s == 0) & (c == 0))
    def _():
        pltpu.sync_copy(dinv, cnt_sh)
    plsc.subcore_barrier()

    @pl.when(c == 0)
    def _():
        def _count_blk(b, carry):
            idx = b * NS + s

            @pl.when(idx < NBLK)
            def _():
                pltpu.sync_copy(row_hbm.at[pl.ds(idx * BLK, BLK)], idx_blk)
                pltpu.sync_copy(ones_blk, cnt_sh.at[idx_blk], add=True)
            return carry
        lax.fori_loop(0, (NBLK + NS - 1) // NS, _count_blk, 0)
    plsc.subcore_barrier()

    @pl.when((s == 0) & (c == 0))
    def _():
        pltpu.sync_copy(cnt_sh, dinv)

        def _mk_dinv(i, carry):
            d = dinv[pl.ds(i * L, L)] + 1.0
            dinv[pl.ds(i * L, L)] = _rsqrt_vec(d)
            return carry
        lax.fori_loop(0, NPAD // L, _mk_dinv, 0)
        pltpu.sync_copy(dinv, dinv_hbm)


_sc_count = pl.kernel(
    _cnt_body,
    out_type=jax.ShapeDtypeStruct((NPAD,), jnp.float32),
    mesh=plsc.VectorSubcoreMesh(core_axis_name="c", subcore_axis_name="s"),
    compiler_params=pltpu.CompilerParams(needs_layout_passes=False),
    scratch_types=[
        pltpu.VMEM((BLK,), jnp.int32),
        pltpu.VMEM((BLK,), jnp.float32),
        pltpu.VMEM((NPAD,), jnp.float32),
        pltpu.VMEM_SHARED((NPAD,), jnp.float32),
    ],
)


def _sc_body(row_hbm, col_hbm, h_hbm, e_hbm, dinv_hbm, out_hbm,
             rb0, rb1, cb0, cb1, dinv, wl_row, wl_eid, wl_col,
             wlb_row, wlb_eid, wlb_col,
             nrm_buf, acc, h_b0, h_b1, e_b0, e_b1,
             sh0, sh1, se0, se1, sr0, sr1, sc0, sc1):
    c = lax.axis_index("c")
    s = lax.axis_index("s")
    wid = s * NC + c
    zf = jnp.zeros((L,), jnp.float32)
    zi = jnp.zeros((L,), jnp.int32)
    iota = lax.iota(jnp.int32, L)
    h_bufs = (h_b0, h_b1)
    e_bufs = (e_b0, e_b1)
    sems_h = (sh0, sh1)
    sems_e = (se0, se1)
    rbufs = (rb0, rb1)
    cbufs = (cb0, cb1)
    sems_r = (sr0, sr1)
    sems_c = (sc0, sc1)

    pltpu.sync_copy(dinv_hbm, dinv.at[pl.ds(0, NPAD)])

    def _start(k, b):
        pltpu.make_async_copy(
            h_hbm.at[wl_row.at[pl.ds(k * K, K)]], h_bufs[b], sems_h[b]).start()
        pltpu.make_async_copy(
            e_hbm.at[wl_eid.at[pl.ds(k * K, K)]], e_bufs[b], sems_e[b]).start()

    def _make_pass(p):
        vbase = wid * NPASS + p
        rbase = vbase * RPP

        def _zero_acc(j, carry):
            acc[pl.ds(j * L, L)] = zf
            return carry
        lax.fori_loop(0, RPP * D // L, _zero_acc, 0)

        def _finish(k, b, lim):
            pltpu.make_async_copy(
                h_hbm.at[wl_row.at[pl.ds(k * K, K)]], h_bufs[b],
                sems_h[b]).wait()
            pltpu.make_async_copy(
                e_hbm.at[wl_eid.at[pl.ds(k * K, K)]], e_bufs[b],
                sems_e[b]).wait()
            nv = lim - k * K
            for t in range(K // L):
                rv = wl_row[pl.ds(k * K + t * L, L)]
                cv = wl_col[pl.ds(k * K + t * L, L)]
                nrm = (plsc.load_gather(dinv, [rv]) *
                       plsc.load_gather(dinv, [cv]))
                valid = (iota + t * L) < nv
                nrm_buf[pl.ds(t * L, L)] = jnp.where(valid, nrm, 0.0)
            h_buf = h_bufs[b]
            e_buf = e_bufs[b]

            def _edge(j, carry):
                nrm = nrm_buf[pl.ds(j, L)][0]
                cl = wl_col[pl.ds(k * K + j, L)][0] - rbase
                base = cl << 8
                for g in range(NV // 4):
                    hv = [h_buf[j, pl.ds((g * 4 + i) * L, L)]
                          for i in range(4)]
                    ev = [e_buf[j, pl.ds((g * 4 + i) * L, L)]
                          for i in range(4)]
                    sv = [hv[i] + ev[i] for i in range(4)]
                    mv = [jnp.maximum(sv[i], 0.0) * nrm for i in range(4)]
                    for i in range(4):
                        plsc.addupdate(
                            acc.at[pl.ds(base + (g * 4 + i) * L, L)], mv[i])
                return carry
            lax.fori_loop(0, K, _edge, 0, unroll=2)

        def _drain(nch, lim):
            @pl.when(0 < nch)
            def _():
                _start(0, 0)
            ngroups = (nch + 1) >> 1

            def _grp(g, carry):
                k0 = g * NB
                for b in range(NB):
                    k = k0 + b
                    nxt = k + NB - 1

                    @pl.when(nxt < nch)
                    def _(nxt=nxt, b=b):
                        _start(nxt, (b + NB - 1) % NB)

                    @pl.when(k < nch)
                    def _(k=k, b=b):
                        _finish(k, b, lim)
                return carry
            lax.fori_loop(0, ngroups, _grp, 0)

        def _scan_pair(bufs, bbase, g, offs):
            offa, offb = offs
            va = 2 * g
            vb = 2 * g + 1
            ra = bufs[0][pl.ds(va * L, L)]
            ca = bufs[1][pl.ds(va * L, L)]
            rb = bufs[0][pl.ds(vb * L, L)]
            cb = bufs[1][pl.ds(vb * L, L)]
            ma = (((ca >> 5) * 6554) >> 15) == vbase
            mb = (((cb >> 5) * 6554) >> 15) == vbase
            ea = (bbase + va * L) + iota
            eb = (bbase + vb * L) + iota
            plsc.store_compressed(wl_row.at[pl.ds(offa, L)], ra, mask=ma)
            plsc.store_compressed(wlb_row.at[pl.ds(offb, L)], rb, mask=mb)
            plsc.store_compressed(wl_eid.at[pl.ds(offa, L)], ea, mask=ma)
            plsc.store_compressed(wlb_eid.at[pl.ds(offb, L)], eb, mask=mb)
            plsc.store_compressed(wl_col.at[pl.ds(offa, L)], ca, mask=ma)
            plsc.store_compressed(wlb_col.at[pl.ds(offb, L)], cb, mask=mb)
            pa = plsc.all_reduce_population_count(ma)[0]
            pb = plsc.all_reduce_population_count(mb)[0]
            return offa + pa, offb + pb

        def _start_blk(b, buf):
            pltpu.make_async_copy(
                row_hbm.at[pl.ds(b * BLK, BLK)], rbufs[buf],
                sems_r[buf]).start()
            pltpu.make_async_copy(
                col_hbm.at[pl.ds(b * BLK, BLK)], cbufs[buf],
                sems_c[buf]).start()

        def _scan_blk(buf, b, off):
            pltpu.make_async_copy(
                row_hbm.at[pl.ds(b * BLK, BLK)], rbufs[buf],
                sems_r[buf]).wait()
            pltpu.make_async_copy(
                col_hbm.at[pl.ds(b * BLK, BLK)], cbufs[buf],
                sems_c[buf]).wait()
            off, offb = lax.fori_loop(
                0, VPB // 2,
                functools.partial(_scan_pair, (rbufs[buf], cbufs[buf]),
                                  b * BLK),
                (off, jnp.int32(0)))

            def _merge(t, carry):
                mr = wlb_row[pl.ds(t * L, L)]
                me = wlb_eid[pl.ds(t * L, L)]
                mc = wlb_col[pl.ds(t * L, L)]
                wl_row[pl.ds(off + t * L, L)] = mr
                wl_eid[pl.ds(off + t * L, L)] = me
                wl_col[pl.ds(off + t * L, L)] = mc
                return carry
            lax.fori_loop(0, (offb + (L - 1)) >> 4, _merge, 0)
            return off + offb

        def _pad_tail(off):
            for t in range(K // L):
                wl_row[pl.ds(off + t * L, L)] = zi
                wl_eid[pl.ds(off + t * L, L)] = zi
                wl_col[pl.ds(off + t * L, L)] = zi + rbase
            return off

        _start_blk(0, 0)

        def _body(g, off):
            for sub in range(NB):
                b = g * NB + sub

                def _real(off, b=b, sub=sub):
                    nxt = b + 1

                    @pl.when(nxt < NBLK)
                    def _():
                        _start_blk(nxt, (sub + 1) % NB)

                    is_last = b == NBLK
                    off = lax.cond(
                        is_last, _pad_tail,
                        functools.partial(_scan_blk, sub, b), off)
                    do = (off >= THRESH) | is_last
                    nch = lax.select(is_last, (off + (K - 1)) >> 6, off >> 6)
                    lim = nch << 6

                    def _do_drain(off):
                        _drain(nch, jnp.minimum(lim, off))
                        base = (off >> 6) << 6
                        for t in range(K // L):
                            lr = wl_row[pl.ds(base + t * L, L)]
                            le = wl_eid[pl.ds(base + t * L, L)]
                            lc = wl_col[pl.ds(base + t * L, L)]
                            wl_row[pl.ds(t * L, L)] = lr
                            wl_eid[pl.ds(t * L, L)] = le
                            wl_col[pl.ds(t * L, L)] = lc
                        return off - base

                    return lax.cond(do, _do_drain, lambda o: o, off)

                off = lax.cond(b <= NBLK, _real, lambda o: o, off)
            return off
        lax.fori_loop(0, (NBLK + 2 + NB - 1) // NB, _body, jnp.int32(0))

        pltpu.sync_copy(acc, out_hbm.at[pl.ds(rbase * D, RPP * D)])

    for p in range(NPASS):
        _make_pass(p)


_sc_main = pl.kernel(
    _sc_body,
    out_type=jax.ShapeDtypeStruct((NPAD * D,), jnp.float32),
    mesh=plsc.VectorSubcoreMesh(core_axis_name="c", subcore_axis_name="s"),
    compiler_params=pltpu.CompilerParams(needs_layout_passes=False),
    scratch_types=[
        pltpu.VMEM((BLK,), jnp.int32),
        pltpu.VMEM((BLK,), jnp.int32),
        pltpu.VMEM((BLK,), jnp.int32),
        pltpu.VMEM((BLK,), jnp.int32),
        pltpu.VMEM((NPAD,), jnp.float32),
        pltpu.VMEM((WLSZ,), jnp.int32),
        pltpu.VMEM((WLSZ,), jnp.int32),
        pltpu.VMEM((WLSZ,), jnp.int32),
        pltpu.VMEM((BLK // 2 + L,), jnp.int32),
        pltpu.VMEM((BLK // 2 + L,), jnp.int32),
        pltpu.VMEM((BLK // 2 + L,), jnp.int32),
        pltpu.VMEM((K + L,), jnp.float32),
        pltpu.VMEM((RPP * D,), jnp.float32),
        pltpu.VMEM((K, D), jnp.float32),
        pltpu.VMEM((K, D), jnp.float32),
        pltpu.VMEM((K, D), jnp.float32),
        pltpu.VMEM((K, D), jnp.float32),
        pltpu.SemaphoreType.DMA,
        pltpu.SemaphoreType.DMA,
        pltpu.SemaphoreType.DMA,
        pltpu.SemaphoreType.DMA,
        pltpu.SemaphoreType.DMA,
        pltpu.SemaphoreType.DMA,
        pltpu.SemaphoreType.DMA,
        pltpu.SemaphoreType.DMA,
    ],
)


def _h_body(x_ref, w_ref, b_ref, o_ref):
    o_ref[...] = lax.dot_general(
        x_ref[...], w_ref[...], (((1,), (1,)), ((), ())),
        preferred_element_type=jnp.float32) + b_ref[...]


_h_call = pl.pallas_call(
    _h_body,
    grid=(5,),
    in_specs=[
        pl.BlockSpec((N // 5, D), lambda i: (i, 0)),
        pl.BlockSpec((D, D), lambda i: (0, 0)),
        pl.BlockSpec((1, D), lambda i: (0, 0)),
    ],
    out_specs=pl.BlockSpec((N // 5, D), lambda i: (i, 0)),
    out_shape=jax.ShapeDtypeStruct((N, D), jnp.float32),
)


def _e_body(a_ref, w_ref, b_ref, o_ref):
    o_ref[...] = lax.dot_general(
        a_ref[...], w_ref[...], (((1,), (0,)), ((), ())),
        preferred_element_type=jnp.float32) + b_ref[...]


_e_call = pl.pallas_call(
    _e_body,
    grid=(80,),
    in_specs=[
        pl.BlockSpec((2000, EDP), lambda i: (i, 0)),
        pl.BlockSpec((EDP, D), lambda i: (0, 0)),
        pl.BlockSpec((1, D), lambda i: (0, 0)),
    ],
    out_specs=pl.BlockSpec((2000, D), lambda i: (i, 0)),
    out_shape=jax.ShapeDtypeStruct((NE, D), jnp.float32),
)


def _fin_body(a_ref, h_ref, d_ref, r_ref, o_ref):
    dd = d_ref[...] * d_ref[...]
    o_ref[...] = a_ref[...] + jnp.maximum(h_ref[...] + r_ref[...], 0.0) * dd


_fin_call = pl.pallas_call(
    _fin_body,
    grid=(5,),
    in_specs=[
        pl.BlockSpec((N // 5, D), lambda i: (i, 0)),
        pl.BlockSpec((N // 5, D), lambda i: (i, 0)),
        pl.BlockSpec((N // 5, 1), lambda i: (i, 0)),
        pl.BlockSpec((1, D), lambda i: (0, 0)),
    ],
    out_specs=pl.BlockSpec((N // 5, D), lambda i: (i, 0)),
    out_shape=jax.ShapeDtypeStruct((N, D), jnp.float32),
)


@jax.jit
def kernel(x, edge_index, edge_attr, W_lin, b_lin, W_edge, b_edge, root_emb):
    ei = edge_index.astype(jnp.int32)
    row = ei[0]
    col = ei[1]
    attr_pad = jnp.pad(edge_attr, ((0, 0), (0, EDP - ED)))
    w_e = jnp.pad(W_edge.T, ((0, EDP - ED), (0, 0)))
    h = _h_call(x, W_lin, b_lin.reshape(1, D))
    e = _e_call(attr_pad, w_e, b_edge.reshape(1, D))
    dinv = _sc_count(row)
    aggr = _sc_main(row, col, h, e, dinv)
    aggr = aggr.reshape(NPAD, D)
    out = _fin_call(aggr[:N], h, dinv[:N].reshape(N, 1),
                    root_emb.reshape(1, D))
    return out


# speedup vs baseline: 1.4861x; 1.0105x over previous
import functools

import jax
import jax.numpy as jnp
from jax import lax
from jax.experimental import pallas as pl
from jax.experimental.pallas import tpu as pltpu
from jax.experimental.pallas import tpu_sc as plsc

N = 10000
NPAD = 10240
NE = 160000
D = 256
ED = 7
EDP = 8

NC, NS, L = 2, 16, 16
NW = NC * NS
NV = D // L
NPASS = 2
RPP = NPAD // NW // NPASS

BLK = 800
VPB = BLK // L
NBLK = NE // BLK

K = 64
NB = 2
THRESH = 512
WLSZ = THRESH + BLK + 80


def _rsqrt_vec(d):
    i = lax.bitcast_convert_type(d, jnp.int32)
    i = jnp.int32(0x5F3759DF) - (i >> 1)
    y = lax.bitcast_convert_type(i, jnp.float32)
    for _ in range(3):
        y = y * (1.5 - 0.5 * d * y * y)
    return y


def _cnt_body(row_hbm, dinv_hbm, idx_blk, ones_blk, dinv, cnt_sh):
    c = lax.axis_index("c")
    s = lax.axis_index("s")
    zf = jnp.zeros((L,), jnp.float32)

    def _fill_ones(i, carry):
        ones_blk[pl.ds(i * L, L)] = zf + 1.0
        return carry
    lax.fori_loop(0, BLK // L, _fill_ones, 0)

    def _zero_dinv(i, carry):
        dinv[pl.ds(i * L, L)] = zf
        return carry
    lax.fori_loop(0, NPAD // L, _zero_dinv, 0)

    @pl.when((s == 0) & (c == 0))
    def _():
        pltpu.sync_copy(dinv, cnt_sh)
    plsc.subcore_barrier()

    @pl.when(c == 0)
    def _():
        def _count_blk(b, carry):
            idx = b * NS + s

            @pl.when(idx < NBLK)
            def _():
                pltpu.sync_copy(row_hbm.at[pl.ds(idx * BLK, BLK)], idx_blk)
                pltpu.sync_copy(ones_blk, cnt_sh.at[idx_blk], add=True)
            return carry
        lax.fori_loop(0, (NBLK + NS - 1) // NS, _count_blk, 0)
    plsc.subcore_barrier()

    @pl.when((s == 0) & (c == 0))
    def _():
        pltpu.sync_copy(cnt_sh, dinv)

        def _mk_dinv(i, carry):
            d = dinv[pl.ds(i * L, L)] + 1.0
            dinv[pl.ds(i * L, L)] = _rsqrt_vec(d)
            return carry
        lax.fori_loop(0, NPAD // L, _mk_dinv, 0)
        pltpu.sync_copy(dinv, dinv_hbm)


_sc_count = pl.kernel(
    _cnt_body,
    out_type=jax.ShapeDtypeStruct((NPAD,), jnp.float32),
    mesh=plsc.VectorSubcoreMesh(core_axis_name="c", subcore_axis_name="s"),
    compiler_params=pltpu.CompilerParams(needs_layout_passes=False),
    scratch_types=[
        pltpu.VMEM((BLK,), jnp.int32),
        pltpu.VMEM((BLK,), jnp.float32),
        pltpu.VMEM((NPAD,), jnp.float32),
        pltpu.VMEM_SHARED((NPAD,), jnp.float32),
    ],
)


def _sc_body(row_hbm, col_hbm, h_hbm, e_hbm, dinv_hbm, out_hbm,
             rb0, rb1, cb0, cb1, dinv, wl_row, wl_eid, wl_col,
             wlb_row, wlb_eid, wlb_col,
             nrm_buf, acc, h_b0, h_b1, e_b0, e_b1,
             sh0, sh1, se0, se1, sr0, sr1, sc0, sc1):
    c = lax.axis_index("c")
    s = lax.axis_index("s")
    wid = s * NC + c
    zf = jnp.zeros((L,), jnp.float32)
    zi = jnp.zeros((L,), jnp.int32)
    iota = lax.iota(jnp.int32, L)
    h_bufs = (h_b0, h_b1)
    e_bufs = (e_b0, e_b1)
    sems_h = (sh0, sh1)
    sems_e = (se0, se1)
    rbufs = (rb0, rb1)
    cbufs = (cb0, cb1)
    sems_r = (sr0, sr1)
    sems_c = (sc0, sc1)

    pltpu.sync_copy(dinv_hbm, dinv.at[pl.ds(0, NPAD)])

    def _start(k, b):
        pltpu.make_async_copy(
            h_hbm.at[wl_row.at[pl.ds(k * K, K)]], h_bufs[b], sems_h[b]).start()
        pltpu.make_async_copy(
            e_hbm.at[wl_eid.at[pl.ds(k * K, K)]], e_bufs[b], sems_e[b]).start()

    def _make_pass(p):
        vbase = wid * NPASS + p
        rbase = vbase * RPP

        def _zero_acc(j, carry):
            acc[pl.ds(j * L, L)] = zf
            return carry
        lax.fori_loop(0, RPP * D // L, _zero_acc, 0)

        def _finish(k, b, lim):
            pltpu.make_async_copy(
                h_hbm.at[wl_row.at[pl.ds(k * K, K)]], h_bufs[b],
                sems_h[b]).wait()
            pltpu.make_async_copy(
                e_hbm.at[wl_eid.at[pl.ds(k * K, K)]], e_bufs[b],
                sems_e[b]).wait()
            nv = lim - k * K
            for t in range(K // L):
                rv = wl_row[pl.ds(k * K + t * L, L)]
                cv = wl_col[pl.ds(k * K + t * L, L)]
                nrm = (plsc.load_gather(dinv, [rv]) *
                       plsc.load_gather(dinv, [cv]))
                valid = (iota + t * L) < nv
                nrm_buf[pl.ds(t * L, L)] = jnp.where(valid, nrm, 0.0)
            h_buf = h_bufs[b]
            e_buf = e_bufs[b]

            def _edge(j, carry):
                nrm = nrm_buf[pl.ds(j, L)][0]
                cl = wl_col[pl.ds(k * K + j, L)][0] - rbase
                base = cl << 8
                for g in range(NV // 4):
                    hv = [h_buf[j, pl.ds((g * 4 + i) * L, L)]
                          for i in range(4)]
                    ev = [e_buf[j, pl.ds((g * 4 + i) * L, L)]
                          for i in range(4)]
                    sv = [hv[i] + ev[i] for i in range(4)]
                    mv = [jnp.maximum(sv[i], 0.0) * nrm for i in range(4)]
                    for i in range(4):
                        plsc.addupdate(
                            acc.at[pl.ds(base + (g * 4 + i) * L, L)], mv[i])
                return carry
            lax.fori_loop(0, K, _edge, 0, unroll=2)

        def _drain(nch, lim):
            @pl.when(0 < nch)
            def _():
                _start(0, 0)
            ngroups = (nch + 1) >> 1

            def _grp(g, carry):
                k0 = g * NB
                for b in range(NB):
                    k = k0 + b
                    nxt = k + NB - 1

                    @pl.when(nxt < nch)
                    def _(nxt=nxt, b=b):
                        _start(nxt, (b + NB - 1) % NB)

                    @pl.when(k < nch)
                    def _(k=k, b=b):
                        _finish(k, b, lim)
                return carry
            lax.fori_loop(0, ngroups, _grp, 0)

        def _scan_pair(bufs, bbase, g, offs):
            offa, offb = offs
            va = 2 * g
            vb = 2 * g + 1
            ra = bufs[0][pl.ds(va * L, L)]
            ca = bufs[1][pl.ds(va * L, L)]
            rb = bufs[0][pl.ds(vb * L, L)]
            cb = bufs[1][pl.ds(vb * L, L)]
            ma = (((ca >> 5) * 6554) >> 15) == vbase
            mb = (((cb >> 5) * 6554) >> 15) == vbase
            ea = (bbase + va * L) + iota
            eb = (bbase + vb * L) + iota
            plsc.store_compressed(wl_row.at[pl.ds(offa, L)], ra, mask=ma)
            plsc.store_compressed(wlb_row.at[pl.ds(offb, L)], rb, mask=mb)
            plsc.store_compressed(wl_eid.at[pl.ds(offa, L)], ea, mask=ma)
            plsc.store_compressed(wlb_eid.at[pl.ds(offb, L)], eb, mask=mb)
            plsc.store_compressed(wl_col.at[pl.ds(offa, L)], ca, mask=ma)
            plsc.store_compressed(wlb_col.at[pl.ds(offb, L)], cb, mask=mb)
            pa = plsc.all_reduce_population_count(ma)[0]
            pb = plsc.all_reduce_population_count(mb)[0]
            return offa + pa, offb + pb

        def _start_blk(b, buf):
            pltpu.make_async_copy(
                row_hbm.at[pl.ds(b * BLK, BLK)], rbufs[buf],
                sems_r[buf]).start()
            pltpu.make_async_copy(
                col_hbm.at[pl.ds(b * BLK, BLK)], cbufs[buf],
                sems_c[buf]).start()

        def _scan_blk(buf, b, off):
            pltpu.make_async_copy(
                row_hbm.at[pl.ds(b * BLK, BLK)], rbufs[buf],
                sems_r[buf]).wait()
            pltpu.make_async_copy(
                col_hbm.at[pl.ds(b * BLK, BLK)], cbufs[buf],
                sems_c[buf]).wait()
            off, offb = lax.fori_loop(
                0, VPB // 2,
                functools.partial(_scan_pair, (rbufs[buf], cbufs[buf]),
                                  b * BLK),
                (off, jnp.int32(0)))

            def _merge(t, carry):
                mr = wlb_row[pl.ds(t * L, L)]
                me = wlb_eid[pl.ds(t * L, L)]
                mc = wlb_col[pl.ds(t * L, L)]
                wl_row[pl.ds(off + t * L, L)] = mr
                wl_eid[pl.ds(off + t * L, L)] = me
                wl_col[pl.ds(off + t * L, L)] = mc
                return carry
            lax.fori_loop(0, (offb + (L - 1)) >> 4, _merge, 0)
            return off + offb

        def _pad_tail(off):
            for t in range(K // L):
                wl_row[pl.ds(off + t * L, L)] = zi
                wl_eid[pl.ds(off + t * L, L)] = zi
                wl_col[pl.ds(off + t * L, L)] = zi + rbase
            return off

        _start_blk(0, 0)

        def _body(g, off):
            for sub in range(NB):
                b = g * NB + sub

                def _real(off, b=b, sub=sub):
                    nxt = b + 1

                    @pl.when(nxt < NBLK)
                    def _():
                        _start_blk(nxt, (sub + 1) % NB)

                    is_last = b == NBLK
                    off = lax.cond(
                        is_last, _pad_tail,
                        functools.partial(_scan_blk, sub, b), off)
                    do = (off >= THRESH) | is_last
                    nch = lax.select(is_last, (off + (K - 1)) >> 6, off >> 6)
                    lim = nch << 6

                    def _do_drain(off):
                        _drain(nch, jnp.minimum(lim, off))
                        base = (off >> 6) << 6
                        for t in range(K // L):
                            lr = wl_row[pl.ds(base + t * L, L)]
                            le = wl_eid[pl.ds(base + t * L, L)]
                            lc = wl_col[pl.ds(base + t * L, L)]
                            wl_row[pl.ds(t * L, L)] = lr
                            wl_eid[pl.ds(t * L, L)] = le
                            wl_col[pl.ds(t * L, L)] = lc
                        return off - base

                    return lax.cond(do, _do_drain, lambda o: o, off)

                off = lax.cond(b <= NBLK, _real, lambda o: o, off)
            return off
        lax.fori_loop(0, (NBLK + 2 + NB - 1) // NB, _body, jnp.int32(0))

        pltpu.sync_copy(acc, out_hbm.at[pl.ds(rbase * D, RPP * D)])

    for p in range(NPASS):
        _make_pass(p)


_sc_main = pl.kernel(
    _sc_body,
    out_type=jax.ShapeDtypeStruct((NPAD * D,), jnp.float32),
    mesh=plsc.VectorSubcoreMesh(core_axis_name="c", subcore_axis_name="s"),
    compiler_params=pltpu.CompilerParams(needs_layout_passes=False),
    scratch_types=[
        pltpu.VMEM((BLK,), jnp.int32),
        pltpu.VMEM((BLK,), jnp.int32),
        pltpu.VMEM((BLK,), jnp.int32),
        pltpu.VMEM((BLK,), jnp.int32),
        pltpu.VMEM((NPAD,), jnp.float32),
        pltpu.VMEM((WLSZ,), jnp.int32),
        pltpu.VMEM((WLSZ,), jnp.int32),
        pltpu.VMEM((WLSZ,), jnp.int32),
        pltpu.VMEM((BLK // 2 + L,), jnp.int32),
        pltpu.VMEM((BLK // 2 + L,), jnp.int32),
        pltpu.VMEM((BLK // 2 + L,), jnp.int32),
        pltpu.VMEM((K + L,), jnp.float32),
        pltpu.VMEM((RPP * D,), jnp.float32),
        pltpu.VMEM((K, D), jnp.float32),
        pltpu.VMEM((K, D), jnp.float32),
        pltpu.VMEM((K, D), jnp.float32),
        pltpu.VMEM((K, D), jnp.float32),
        pltpu.SemaphoreType.DMA,
        pltpu.SemaphoreType.DMA,
        pltpu.SemaphoreType.DMA,
        pltpu.SemaphoreType.DMA,
        pltpu.SemaphoreType.DMA,
        pltpu.SemaphoreType.DMA,
        pltpu.SemaphoreType.DMA,
        pltpu.SemaphoreType.DMA,
    ],
)


def _h_body(x_ref, w_ref, b_ref, o_ref):
    o_ref[...] = lax.dot_general(
        x_ref[...], w_ref[...], (((1,), (1,)), ((), ())),
        preferred_element_type=jnp.float32) + b_ref[...]


_h_call = pl.pallas_call(
    _h_body,
    grid=(5,),
    in_specs=[
        pl.BlockSpec((N // 5, D), lambda i: (i, 0)),
        pl.BlockSpec((D, D), lambda i: (0, 0)),
        pl.BlockSpec((1, D), lambda i: (0, 0)),
    ],
    out_specs=pl.BlockSpec((N // 5, D), lambda i: (i, 0)),
    out_shape=jax.ShapeDtypeStruct((N, D), jnp.float32),
)


def _e_body(a_ref, w_ref, b_ref, o_ref):
    o_ref[...] = lax.dot_general(
        a_ref[...], w_ref[...], (((1,), (0,)), ((), ())),
        preferred_element_type=jnp.float32) + b_ref[...]


_e_call = pl.pallas_call(
    _e_body,
    grid=(80,),
    in_specs=[
        pl.BlockSpec((2000, EDP), lambda i: (i, 0)),
        pl.BlockSpec((EDP, D), lambda i: (0, 0)),
        pl.BlockSpec((1, D), lambda i: (0, 0)),
    ],
    out_specs=pl.BlockSpec((2000, D), lambda i: (i, 0)),
    out_shape=jax.ShapeDtypeStruct((NE, D), jnp.float32),
)


def _fin_body(a_ref, h_ref, d_ref, r_ref, o_ref):
    dd = d_ref[...] * d_ref[...]
    o_ref[...] = a_ref[...] + jnp.maximum(h_ref[...] + r_ref[...], 0.0) * dd


_fin_call = pl.pallas_call(
    _fin_body,
    grid=(5,),
    in_specs=[
        pl.BlockSpec((N // 5, D), lambda i: (i, 0)),
        pl.BlockSpec((N // 5, D), lambda i: (i, 0)),
        pl.BlockSpec((N // 5, 1), lambda i: (i, 0)),
        pl.BlockSpec((1, D), lambda i: (0, 0)),
    ],
    out_specs=pl.BlockSpec((N // 5, D), lambda i: (i, 0)),
    out_shape=jax.ShapeDtypeStruct((N, D), jnp.float32),
)


@jax.jit
def kernel(x, edge_index, edge_attr, W_lin, b_lin, W_edge, b_edge, root_emb):
    ei = edge_index.astype(jnp.int32)
    row = ei[0]
    col = ei[1]
    attr_pad = jnp.pad(edge_attr, ((0, 0), (0, EDP - ED)))
    w_e = jnp.pad(W_edge.T, ((0, EDP - ED), (0, 0)))
    h = _h_call(x, W_lin, b_lin.reshape(1, D))
    e = _e_call(attr_pad, w_e, b_edge.reshape(1, D))
    dinv = _sc_count(row)
    aggr = _sc_main(row, col, h, e, dinv)
    out = _fin_call(aggr.reshape(NPAD, D), h, dinv.reshape(NPAD, 1),
                    root_emb.reshape(1, D))
    return out


# speedup vs baseline: 1.5124x; 1.0177x over previous
import functools

import jax
import jax.numpy as jnp
from jax import lax
from jax.experimental import pallas as pl
from jax.experimental.pallas import tpu as pltpu
from jax.experimental.pallas import tpu_sc as plsc

N = 10000
NPAD = 10240
NE = 160000
D = 256
ED = 7
EDP = 8

NC, NS, L = 2, 16, 16
NW = NC * NS
NV = D // L
NPASS = 2
RPP = NPAD // NW // NPASS

BLK = 800
VPB = BLK // L
NBLK = NE // BLK

K = 64
NB = 2
THRESH = 768
WLSZ = THRESH + BLK + 80


def _rsqrt_vec(d):
    i = lax.bitcast_convert_type(d, jnp.int32)
    i = jnp.int32(0x5F3759DF) - (i >> 1)
    y = lax.bitcast_convert_type(i, jnp.float32)
    for _ in range(3):
        y = y * (1.5 - 0.5 * d * y * y)
    return y


def _cnt_body(row_hbm, dinv_hbm, idx_blk, ones_blk, dinv, cnt_sh):
    c = lax.axis_index("c")
    s = lax.axis_index("s")
    zf = jnp.zeros((L,), jnp.float32)

    def _fill_ones(i, carry):
        ones_blk[pl.ds(i * L, L)] = zf + 1.0
        return carry
    lax.fori_loop(0, BLK // L, _fill_ones, 0)

    def _zero_dinv(i, carry):
        dinv[pl.ds(i * L, L)] = zf
        return carry
    lax.fori_loop(0, NPAD // L, _zero_dinv, 0)

    @pl.when((s == 0) & (c == 0))
    def _():
        pltpu.sync_copy(dinv, cnt_sh)
    plsc.subcore_barrier()

    @pl.when(c == 0)
    def _():
        def _count_blk(b, carry):
            idx = b * NS + s

            @pl.when(idx < NBLK)
            def _():
                pltpu.sync_copy(row_hbm.at[pl.ds(idx * BLK, BLK)], idx_blk)
                pltpu.sync_copy(ones_blk, cnt_sh.at[idx_blk], add=True)
            return carry
        lax.fori_loop(0, (NBLK + NS - 1) // NS, _count_blk, 0)
    plsc.subcore_barrier()

    @pl.when((s == 0) & (c == 0))
    def _():
        pltpu.sync_copy(cnt_sh, dinv)

        def _mk_dinv(i, carry):
            d = dinv[pl.ds(i * L, L)] + 1.0
            dinv[pl.ds(i * L, L)] = _rsqrt_vec(d)
            return carry
        lax.fori_loop(0, NPAD // L, _mk_dinv, 0)
        pltpu.sync_copy(dinv, dinv_hbm)


_sc_count = pl.kernel(
    _cnt_body,
    out_type=jax.ShapeDtypeStruct((NPAD,), jnp.float32),
    mesh=plsc.VectorSubcoreMesh(core_axis_name="c", subcore_axis_name="s"),
    compiler_params=pltpu.CompilerParams(needs_layout_passes=False),
    scratch_types=[
        pltpu.VMEM((BLK,), jnp.int32),
        pltpu.VMEM((BLK,), jnp.float32),
        pltpu.VMEM((NPAD,), jnp.float32),
        pltpu.VMEM_SHARED((NPAD,), jnp.float32),
    ],
)


def _sc_body(row_hbm, col_hbm, h_hbm, e_hbm, dinv_hbm, out_hbm,
             rb0, rb1, cb0, cb1, dinv, wl_row, wl_eid, wl_col,
             wlb_row, wlb_eid, wlb_col,
             nrm_buf, acc, h_b0, h_b1, e_b0, e_b1,
             sh0, sh1, se0, se1, sr0, sr1, sc0, sc1):
    c = lax.axis_index("c")
    s = lax.axis_index("s")
    wid = s * NC + c
    zf = jnp.zeros((L,), jnp.float32)
    zi = jnp.zeros((L,), jnp.int32)
    iota = lax.iota(jnp.int32, L)
    h_bufs = (h_b0, h_b1)
    e_bufs = (e_b0, e_b1)
    sems_h = (sh0, sh1)
    sems_e = (se0, se1)
    rbufs = (rb0, rb1)
    cbufs = (cb0, cb1)
    sems_r = (sr0, sr1)
    sems_c = (sc0, sc1)

    pltpu.sync_copy(dinv_hbm, dinv.at[pl.ds(0, NPAD)])

    def _start(k, b):
        pltpu.make_async_copy(
            h_hbm.at[wl_row.at[pl.ds(k * K, K)]], h_bufs[b], sems_h[b]).start()
        pltpu.make_async_copy(
            e_hbm.at[wl_eid.at[pl.ds(k * K, K)]], e_bufs[b], sems_e[b]).start()

    def _make_pass(p):
        vbase = wid * NPASS + p
        rbase = vbase * RPP

        def _zero_acc(j, carry):
            acc[pl.ds(j * L, L)] = zf
            return carry
        lax.fori_loop(0, RPP * D // L, _zero_acc, 0)

        def _finish(k, b, lim):
            pltpu.make_async_copy(
                h_hbm.at[wl_row.at[pl.ds(k * K, K)]], h_bufs[b],
                sems_h[b]).wait()
            pltpu.make_async_copy(
                e_hbm.at[wl_eid.at[pl.ds(k * K, K)]], e_bufs[b],
                sems_e[b]).wait()
            nv = lim - k * K
            for t in range(K // L):
                rv = wl_row[pl.ds(k * K + t * L, L)]
                cv = wl_col[pl.ds(k * K + t * L, L)]
                nrm = (plsc.load_gather(dinv, [rv]) *
                       plsc.load_gather(dinv, [cv]))
                valid = (iota + t * L) < nv
                nrm_buf[pl.ds(t * L, L)] = jnp.where(valid, nrm, 0.0)
            h_buf = h_bufs[b]
            e_buf = e_bufs[b]

            def _edge(j, carry):
                nrm = nrm_buf[pl.ds(j, L)][0]
                cl = wl_col[pl.ds(k * K + j, L)][0] - rbase
                base = cl << 8
                for g in range(NV // 4):
                    hv = [h_buf[j, pl.ds((g * 4 + i) * L, L)]
                          for i in range(4)]
                    ev = [e_buf[j, pl.ds((g * 4 + i) * L, L)]
                          for i in range(4)]
                    sv = [hv[i] + ev[i] for i in range(4)]
                    mv = [jnp.maximum(sv[i], 0.0) * nrm for i in range(4)]
                    for i in range(4):
                        plsc.addupdate(
                            acc.at[pl.ds(base + (g * 4 + i) * L, L)], mv[i])
                return carry
            lax.fori_loop(0, K, _edge, 0, unroll=4)

        def _drain(nch, lim):
            @pl.when(0 < nch)
            def _():
                _start(0, 0)
            ngroups = (nch + 1) >> 1

            def _grp(g, carry):
                k0 = g * NB
                for b in range(NB):
                    k = k0 + b
                    nxt = k + NB - 1

                    @pl.when(nxt < nch)
                    def _(nxt=nxt, b=b):
                        _start(nxt, (b + NB - 1) % NB)

                    @pl.when(k < nch)
                    def _(k=k, b=b):
                        _finish(k, b, lim)
                return carry
            lax.fori_loop(0, ngroups, _grp, 0)

        def _scan_pair(bufs, bbase, g, offs):
            offa, offb = offs
            va = 2 * g
            vb = 2 * g + 1
            ra = bufs[0][pl.ds(va * L, L)]
            ca = bufs[1][pl.ds(va * L, L)]
            rb = bufs[0][pl.ds(vb * L, L)]
            cb = bufs[1][pl.ds(vb * L, L)]
            ma = (((ca >> 5) * 6554) >> 15) == vbase
            mb = (((cb >> 5) * 6554) >> 15) == vbase
            ea = (bbase + va * L) + iota
            eb = (bbase + vb * L) + iota
            plsc.store_compressed(wl_row.at[pl.ds(offa, L)], ra, mask=ma)
            plsc.store_compressed(wlb_row.at[pl.ds(offb, L)], rb, mask=mb)
            plsc.store_compressed(wl_eid.at[pl.ds(offa, L)], ea, mask=ma)
            plsc.store_compressed(wlb_eid.at[pl.ds(offb, L)], eb, mask=mb)
            plsc.store_compressed(wl_col.at[pl.ds(offa, L)], ca, mask=ma)
            plsc.store_compressed(wlb_col.at[pl.ds(offb, L)], cb, mask=mb)
            pa = plsc.all_reduce_population_count(ma)[0]
            pb = plsc.all_reduce_population_count(mb)[0]
            return offa + pa, offb + pb

        def _start_blk(b, buf):
            pltpu.make_async_copy(
                row_hbm.at[pl.ds(b * BLK, BLK)], rbufs[buf],
                sems_r[buf]).start()
            pltpu.make_async_copy(
                col_hbm.at[pl.ds(b * BLK, BLK)], cbufs[buf],
                sems_c[buf]).start()

        def _scan_blk(buf, b, off):
            pltpu.make_async_copy(
                row_hbm.at[pl.ds(b * BLK, BLK)], rbufs[buf],
                sems_r[buf]).wait()
            pltpu.make_async_copy(
                col_hbm.at[pl.ds(b * BLK, BLK)], cbufs[buf],
                sems_c[buf]).wait()
            off, offb = lax.fori_loop(
                0, VPB // 2,
                functools.partial(_scan_pair, (rbufs[buf], cbufs[buf]),
                                  b * BLK),
                (off, jnp.int32(0)))

            def _merge(t, carry):
                mr = wlb_row[pl.ds(t * L, L)]
                me = wlb_eid[pl.ds(t * L, L)]
                mc = wlb_col[pl.ds(t * L, L)]
                wl_row[pl.ds(off + t * L, L)] = mr
                wl_eid[pl.ds(off + t * L, L)] = me
                wl_col[pl.ds(off + t * L, L)] = mc
                return carry
            lax.fori_loop(0, (offb + (L - 1)) >> 4, _merge, 0)
            return off + offb

        def _pad_tail(off):
            for t in range(K // L):
                wl_row[pl.ds(off + t * L, L)] = zi
                wl_eid[pl.ds(off + t * L, L)] = zi
                wl_col[pl.ds(off + t * L, L)] = zi + rbase
            return off

        _start_blk(0, 0)

        def _body(g, off):
            for sub in range(NB):
                b = g * NB + sub

                def _real(off, b=b, sub=sub):
                    nxt = b + 1

                    @pl.when(nxt < NBLK)
                    def _():
                        _start_blk(nxt, (sub + 1) % NB)

                    is_last = b == NBLK
                    off = lax.cond(
                        is_last, _pad_tail,
                        functools.partial(_scan_blk, sub, b), off)
                    do = (off >= THRESH) | is_last
                    nch = lax.select(is_last, (off + (K - 1)) >> 6, off >> 6)
                    lim = nch << 6

                    def _do_drain(off):
                        _drain(nch, jnp.minimum(lim, off))
                        base = (off >> 6) << 6
                        for t in range(K // L):
                            lr = wl_row[pl.ds(base + t * L, L)]
                            le = wl_eid[pl.ds(base + t * L, L)]
                            lc = wl_col[pl.ds(base + t * L, L)]
                            wl_row[pl.ds(t * L, L)] = lr
                            wl_eid[pl.ds(t * L, L)] = le
                            wl_col[pl.ds(t * L, L)] = lc
                        return off - base

                    return lax.cond(do, _do_drain, lambda o: o, off)

                off = lax.cond(b <= NBLK, _real, lambda o: o, off)
            return off
        lax.fori_loop(0, (NBLK + 2 + NB - 1) // NB, _body, jnp.int32(0))

        pltpu.sync_copy(acc, out_hbm.at[pl.ds(rbase * D, RPP * D)])

    for p in range(NPASS):
        _make_pass(p)


_sc_main = pl.kernel(
    _sc_body,
    out_type=jax.ShapeDtypeStruct((NPAD * D,), jnp.float32),
    mesh=plsc.VectorSubcoreMesh(core_axis_name="c", subcore_axis_name="s"),
    compiler_params=pltpu.CompilerParams(needs_layout_passes=False),
    scratch_types=[
        pltpu.VMEM((BLK,), jnp.int32),
        pltpu.VMEM((BLK,), jnp.int32),
        pltpu.VMEM((BLK,), jnp.int32),
        pltpu.VMEM((BLK,), jnp.int32),
        pltpu.VMEM((NPAD,), jnp.float32),
        pltpu.VMEM((WLSZ,), jnp.int32),
        pltpu.VMEM((WLSZ,), jnp.int32),
        pltpu.VMEM((WLSZ,), jnp.int32),
        pltpu.VMEM((BLK // 2 + L,), jnp.int32),
        pltpu.VMEM((BLK // 2 + L,), jnp.int32),
        pltpu.VMEM((BLK // 2 + L,), jnp.int32),
        pltpu.VMEM((K + L,), jnp.float32),
        pltpu.VMEM((RPP * D,), jnp.float32),
        pltpu.VMEM((K, D), jnp.float32),
        pltpu.VMEM((K, D), jnp.float32),
        pltpu.VMEM((K, D), jnp.float32),
        pltpu.VMEM((K, D), jnp.float32),
        pltpu.SemaphoreType.DMA,
        pltpu.SemaphoreType.DMA,
        pltpu.SemaphoreType.DMA,
        pltpu.SemaphoreType.DMA,
        pltpu.SemaphoreType.DMA,
        pltpu.SemaphoreType.DMA,
        pltpu.SemaphoreType.DMA,
        pltpu.SemaphoreType.DMA,
    ],
)


def _h_body(x_ref, w_ref, b_ref, o_ref):
    o_ref[...] = lax.dot_general(
        x_ref[...], w_ref[...], (((1,), (1,)), ((), ())),
        preferred_element_type=jnp.float32) + b_ref[...]


_h_call = pl.pallas_call(
    _h_body,
    grid=(5,),
    in_specs=[
        pl.BlockSpec((N // 5, D), lambda i: (i, 0)),
        pl.BlockSpec((D, D), lambda i: (0, 0)),
        pl.BlockSpec((1, D), lambda i: (0, 0)),
    ],
    out_specs=pl.BlockSpec((N // 5, D), lambda i: (i, 0)),
    out_shape=jax.ShapeDtypeStruct((N, D), jnp.float32),
)


def _e_body(a_ref, w_ref, b_ref, o_ref):
    o_ref[...] = lax.dot_general(
        a_ref[...], w_ref[...], (((1,), (0,)), ((), ())),
        preferred_element_type=jnp.float32) + b_ref[...]


_e_call = pl.pallas_call(
    _e_body,
    grid=(80,),
    in_specs=[
        pl.BlockSpec((2000, EDP), lambda i: (i, 0)),
        pl.BlockSpec((EDP, D), lambda i: (0, 0)),
        pl.BlockSpec((1, D), lambda i: (0, 0)),
    ],
    out_specs=pl.BlockSpec((2000, D), lambda i: (i, 0)),
    out_shape=jax.ShapeDtypeStruct((NE, D), jnp.float32),
)


def _fin_body(a_ref, h_ref, d_ref, r_ref, o_ref):
    dd = d_ref[...] * d_ref[...]
    o_ref[...] = a_ref[...] + jnp.maximum(h_ref[...] + r_ref[...], 0.0) * dd


_fin_call = pl.pallas_call(
    _fin_body,
    grid=(5,),
    in_specs=[
        pl.BlockSpec((N // 5, D), lambda i: (i, 0)),
        pl.BlockSpec((N // 5, D), lambda i: (i, 0)),
        pl.BlockSpec((N // 5, 1), lambda i: (i, 0)),
        pl.BlockSpec((1, D), lambda i: (0, 0)),
    ],
    out_specs=pl.BlockSpec((N // 5, D), lambda i: (i, 0)),
    out_shape=jax.ShapeDtypeStruct((N, D), jnp.float32),
)


@jax.jit
def kernel(x, edge_index, edge_attr, W_lin, b_lin, W_edge, b_edge, root_emb):
    ei = edge_index.astype(jnp.int32)
    row = ei[0]
    col = ei[1]
    attr_pad = jnp.pad(edge_attr, ((0, 0), (0, EDP - ED)))
    w_e = jnp.pad(W_edge.T, ((0, EDP - ED), (0, 0)))
    h = _h_call(x, W_lin, b_lin.reshape(1, D))
    e = _e_call(attr_pad, w_e, b_edge.reshape(1, D))
    dinv = _sc_count(row)
    aggr = _sc_main(row, col, h, e, dinv)
    out = _fin_call(aggr.reshape(NPAD, D), h, dinv.reshape(NPAD, 1),
                    root_emb.reshape(1, D))
    return out


# speedup vs baseline: 1.5600x; 1.0315x over previous
import functools

import jax
import jax.numpy as jnp
from jax import lax
from jax.experimental import pallas as pl
from jax.experimental.pallas import tpu as pltpu
from jax.experimental.pallas import tpu_sc as plsc

N = 10000
NPAD = 10240
NE = 160000
D = 256
ED = 7
EDP = 8

NC, NS, L = 2, 16, 16
NW = NC * NS
NV = D // L
NPASS = 2
RPP = NPAD // NW // NPASS

BLK = 800
VPB = BLK // L
NBLK = NE // BLK

K = 64
NB = 2
THRESH = 1024
WLSZ = THRESH + BLK + 80


def _rsqrt_vec(d):
    i = lax.bitcast_convert_type(d, jnp.int32)
    i = jnp.int32(0x5F3759DF) - (i >> 1)
    y = lax.bitcast_convert_type(i, jnp.float32)
    for _ in range(3):
        y = y * (1.5 - 0.5 * d * y * y)
    return y


def _cnt_body(row_hbm, dinv_hbm, idx_blk, ones_blk, dinv, cnt_sh):
    c = lax.axis_index("c")
    s = lax.axis_index("s")
    zf = jnp.zeros((L,), jnp.float32)

    def _fill_ones(i, carry):
        ones_blk[pl.ds(i * L, L)] = zf + 1.0
        return carry
    lax.fori_loop(0, BLK // L, _fill_ones, 0)

    def _zero_dinv(i, carry):
        dinv[pl.ds(i * L, L)] = zf
        return carry
    lax.fori_loop(0, NPAD // L, _zero_dinv, 0)

    @pl.when((s == 0) & (c == 0))
    def _():
        pltpu.sync_copy(dinv, cnt_sh)
    plsc.subcore_barrier()

    @pl.when(c == 0)
    def _():
        def _count_blk(b, carry):
            idx = b * NS + s

            @pl.when(idx < NBLK)
            def _():
                pltpu.sync_copy(row_hbm.at[pl.ds(idx * BLK, BLK)], idx_blk)
                pltpu.sync_copy(ones_blk, cnt_sh.at[idx_blk], add=True)
            return carry
        lax.fori_loop(0, (NBLK + NS - 1) // NS, _count_blk, 0)
    plsc.subcore_barrier()

    @pl.when((s == 0) & (c == 0))
    def _():
        pltpu.sync_copy(cnt_sh, dinv)

        def _mk_dinv(i, carry):
            d = dinv[pl.ds(i * L, L)] + 1.0
            dinv[pl.ds(i * L, L)] = _rsqrt_vec(d)
            return carry
        lax.fori_loop(0, NPAD // L, _mk_dinv, 0)
        pltpu.sync_copy(dinv, dinv_hbm)


_sc_count = pl.kernel(
    _cnt_body,
    out_type=jax.ShapeDtypeStruct((NPAD,), jnp.float32),
    mesh=plsc.VectorSubcoreMesh(core_axis_name="c", subcore_axis_name="s"),
    compiler_params=pltpu.CompilerParams(needs_layout_passes=False),
    scratch_types=[
        pltpu.VMEM((BLK,), jnp.int32),
        pltpu.VMEM((BLK,), jnp.float32),
        pltpu.VMEM((NPAD,), jnp.float32),
        pltpu.VMEM_SHARED((NPAD,), jnp.float32),
    ],
)


def _sc_body(row_hbm, col_hbm, h_hbm, e_hbm, dinv_hbm, out_hbm,
             rb0, rb1, cb0, cb1, dinv, wl_row, wl_eid, wl_col,
             wlb_row, wlb_eid, wlb_col,
             nrm_buf, acc, h_b0, h_b1, e_b0, e_b1,
             sh0, sh1, se0, se1, sr0, sr1, sc0, sc1):
    c = lax.axis_index("c")
    s = lax.axis_index("s")
    wid = s * NC + c
    zf = jnp.zeros((L,), jnp.float32)
    zi = jnp.zeros((L,), jnp.int32)
    iota = lax.iota(jnp.int32, L)
    h_bufs = (h_b0, h_b1)
    e_bufs = (e_b0, e_b1)
    sems_h = (sh0, sh1)
    sems_e = (se0, se1)
    rbufs = (rb0, rb1)
    cbufs = (cb0, cb1)
    sems_r = (sr0, sr1)
    sems_c = (sc0, sc1)

    pltpu.sync_copy(dinv_hbm, dinv.at[pl.ds(0, NPAD)])

    def _start(k, b):
        pltpu.make_async_copy(
            h_hbm.at[wl_row.at[pl.ds(k * K, K)]], h_bufs[b], sems_h[b]).start()
        pltpu.make_async_copy(
            e_hbm.at[wl_eid.at[pl.ds(k * K, K)]], e_bufs[b], sems_e[b]).start()

    def _make_pass(p):
        vbase = wid * NPASS + p
        rbase = vbase * RPP

        def _zero_acc(j, carry):
            acc[pl.ds(j * L, L)] = zf
            return carry
        lax.fori_loop(0, RPP * D // L, _zero_acc, 0)

        def _finish(k, b, lim):
            pltpu.make_async_copy(
                h_hbm.at[wl_row.at[pl.ds(k * K, K)]], h_bufs[b],
                sems_h[b]).wait()
            pltpu.make_async_copy(
                e_hbm.at[wl_eid.at[pl.ds(k * K, K)]], e_bufs[b],
                sems_e[b]).wait()
            nv = lim - k * K
            for t in range(K // L):
                rv = wl_row[pl.ds(k * K + t * L, L)]
                cv = wl_col[pl.ds(k * K + t * L, L)]
                nrm = (plsc.load_gather(dinv, [rv]) *
                       plsc.load_gather(dinv, [cv]))
                valid = (iota + t * L) < nv
                nrm_buf[pl.ds(t * L, L)] = jnp.where(valid, nrm, 0.0)
            h_buf = h_bufs[b]
            e_buf = e_bufs[b]

            def _edge(j, carry):
                nrm = nrm_buf[pl.ds(j, L)][0]
                cl = wl_col[pl.ds(k * K + j, L)][0] - rbase
                base = cl << 8
                for g in range(NV // 4):
                    hv = [h_buf[j, pl.ds((g * 4 + i) * L, L)]
                          for i in range(4)]
                    ev = [e_buf[j, pl.ds((g * 4 + i) * L, L)]
                          for i in range(4)]
                    sv = [hv[i] + ev[i] for i in range(4)]
                    mv = [jnp.maximum(sv[i], 0.0) * nrm for i in range(4)]
                    for i in range(4):
                        plsc.addupdate(
                            acc.at[pl.ds(base + (g * 4 + i) * L, L)], mv[i])
                return carry
            lax.fori_loop(0, K, _edge, 0, unroll=4)

        def _drain(nch, lim):
            @pl.when(0 < nch)
            def _():
                _start(0, 0)
            ngroups = (nch + 1) >> 1

            def _grp(g, carry):
                k0 = g * NB
                for b in range(NB):
                    k = k0 + b
                    nxt = k + NB - 1

                    @pl.when(nxt < nch)
                    def _(nxt=nxt, b=b):
                        _start(nxt, (b + NB - 1) % NB)

                    @pl.when(k < nch)
                    def _(k=k, b=b):
                        _finish(k, b, lim)
                return carry
            lax.fori_loop(0, ngroups, _grp, 0)

        def _scan_pair(bufs, bbase, g, offs):
            offa, offb = offs
            va = 2 * g
            vb = 2 * g + 1
            ra = bufs[0][pl.ds(va * L, L)]
            ca = bufs[1][pl.ds(va * L, L)]
            rb = bufs[0][pl.ds(vb * L, L)]
            cb = bufs[1][pl.ds(vb * L, L)]
            ma = (((ca >> 5) * 6554) >> 15) == vbase
            mb = (((cb >> 5) * 6554) >> 15) == vbase
            ea = (bbase + va * L) + iota
            eb = (bbase + vb * L) + iota
            plsc.store_compressed(wl_row.at[pl.ds(offa, L)], ra, mask=ma)
            plsc.store_compressed(wlb_row.at[pl.ds(offb, L)], rb, mask=mb)
            plsc.store_compressed(wl_eid.at[pl.ds(offa, L)], ea, mask=ma)
            plsc.store_compressed(wlb_eid.at[pl.ds(offb, L)], eb, mask=mb)
            plsc.store_compressed(wl_col.at[pl.ds(offa, L)], ca, mask=ma)
            plsc.store_compressed(wlb_col.at[pl.ds(offb, L)], cb, mask=mb)
            pa = plsc.all_reduce_population_count(ma)[0]
            pb = plsc.all_reduce_population_count(mb)[0]
            return offa + pa, offb + pb

        def _start_blk(b, buf):
            pltpu.make_async_copy(
                row_hbm.at[pl.ds(b * BLK, BLK)], rbufs[buf],
                sems_r[buf]).start()
            pltpu.make_async_copy(
                col_hbm.at[pl.ds(b * BLK, BLK)], cbufs[buf],
                sems_c[buf]).start()

        def _scan_blk(buf, b, off):
            pltpu.make_async_copy(
                row_hbm.at[pl.ds(b * BLK, BLK)], rbufs[buf],
                sems_r[buf]).wait()
            pltpu.make_async_copy(
                col_hbm.at[pl.ds(b * BLK, BLK)], cbufs[buf],
                sems_c[buf]).wait()
            off, offb = lax.fori_loop(
                0, VPB // 2,
                functools.partial(_scan_pair, (rbufs[buf], cbufs[buf]),
                                  b * BLK),
                (off, jnp.int32(0)))

            def _merge(t, carry):
                mr = wlb_row[pl.ds(t * L, L)]
                me = wlb_eid[pl.ds(t * L, L)]
                mc = wlb_col[pl.ds(t * L, L)]
                wl_row[pl.ds(off + t * L, L)] = mr
                wl_eid[pl.ds(off + t * L, L)] = me
                wl_col[pl.ds(off + t * L, L)] = mc
                return carry
            lax.fori_loop(0, (offb + (L - 1)) >> 4, _merge, 0)
            return off + offb

        def _pad_tail(off):
            for t in range(K // L):
                wl_row[pl.ds(off + t * L, L)] = zi
                wl_eid[pl.ds(off + t * L, L)] = zi
                wl_col[pl.ds(off + t * L, L)] = zi + rbase
            return off

        _start_blk(0, 0)

        def _body(g, off):
            for sub in range(NB):
                b = g * NB + sub

                def _real(off, b=b, sub=sub):
                    nxt = b + 1

                    @pl.when(nxt < NBLK)
                    def _():
                        _start_blk(nxt, (sub + 1) % NB)

                    is_last = b == NBLK
                    off = lax.cond(
                        is_last, _pad_tail,
                        functools.partial(_scan_blk, sub, b), off)
                    do = (off >= THRESH) | is_last
                    nch = lax.select(is_last, (off + (K - 1)) >> 6, off >> 6)
                    lim = nch << 6

                    def _do_drain(off):
                        _drain(nch, jnp.minimum(lim, off))
                        base = (off >> 6) << 6
                        for t in range(K // L):
                            lr = wl_row[pl.ds(base + t * L, L)]
                            le = wl_eid[pl.ds(base + t * L, L)]
                            lc = wl_col[pl.ds(base + t * L, L)]
                            wl_row[pl.ds(t * L, L)] = lr
                            wl_eid[pl.ds(t * L, L)] = le
                            wl_col[pl.ds(t * L, L)] = lc
                        return off - base

                    return lax.cond(do, _do_drain, lambda o: o, off)

                off = lax.cond(b <= NBLK, _real, lambda o: o, off)
            return off
        lax.fori_loop(0, (NBLK + 2 + NB - 1) // NB, _body, jnp.int32(0))

        pltpu.sync_copy(acc, out_hbm.at[pl.ds(rbase * D, RPP * D)])

    for p in range(NPASS):
        _make_pass(p)


_sc_main = pl.kernel(
    _sc_body,
    out_type=jax.ShapeDtypeStruct((NPAD * D,), jnp.float32),
    mesh=plsc.VectorSubcoreMesh(core_axis_name="c", subcore_axis_name="s"),
    compiler_params=pltpu.CompilerParams(needs_layout_passes=False),
    scratch_types=[
        pltpu.VMEM((BLK,), jnp.int32),
        pltpu.VMEM((BLK,), jnp.int32),
        pltpu.VMEM((BLK,), jnp.int32),
        pltpu.VMEM((BLK,), jnp.int32),
        pltpu.VMEM((NPAD,), jnp.float32),
        pltpu.VMEM((WLSZ,), jnp.int32),
        pltpu.VMEM((WLSZ,), jnp.int32),
        pltpu.VMEM((WLSZ,), jnp.int32),
        pltpu.VMEM((BLK // 2 + L,), jnp.int32),
        pltpu.VMEM((BLK // 2 + L,), jnp.int32),
        pltpu.VMEM((BLK // 2 + L,), jnp.int32),
        pltpu.VMEM((K + L,), jnp.float32),
        pltpu.VMEM((RPP * D,), jnp.float32),
        pltpu.VMEM((K, D), jnp.float32),
        pltpu.VMEM((K, D), jnp.float32),
        pltpu.VMEM((K, D), jnp.float32),
        pltpu.VMEM((K, D), jnp.float32),
        pltpu.SemaphoreType.DMA,
        pltpu.SemaphoreType.DMA,
        pltpu.SemaphoreType.DMA,
        pltpu.SemaphoreType.DMA,
        pltpu.SemaphoreType.DMA,
        pltpu.SemaphoreType.DMA,
        pltpu.SemaphoreType.DMA,
        pltpu.SemaphoreType.DMA,
    ],
)


def _h_body(x_ref, w_ref, b_ref, o_ref):
    o_ref[...] = lax.dot_general(
        x_ref[...], w_ref[...], (((1,), (1,)), ((), ())),
        preferred_element_type=jnp.float32) + b_ref[...]


_h_call = pl.pallas_call(
    _h_body,
    grid=(5,),
    in_specs=[
        pl.BlockSpec((N // 5, D), lambda i: (i, 0)),
        pl.BlockSpec((D, D), lambda i: (0, 0)),
        pl.BlockSpec((1, D), lambda i: (0, 0)),
    ],
    out_specs=pl.BlockSpec((N // 5, D), lambda i: (i, 0)),
    out_shape=jax.ShapeDtypeStruct((N, D), jnp.float32),
)


def _e_body(a_ref, w_ref, b_ref, o_ref):
    o_ref[...] = lax.dot_general(
        a_ref[...], w_ref[...], (((1,), (0,)), ((), ())),
        preferred_element_type=jnp.float32) + b_ref[...]


_e_call = pl.pallas_call(
    _e_body,
    grid=(40,),
    in_specs=[
        pl.BlockSpec((4000, EDP), lambda i: (i, 0)),
        pl.BlockSpec((EDP, D), lambda i: (0, 0)),
        pl.BlockSpec((1, D), lambda i: (0, 0)),
    ],
    out_specs=pl.BlockSpec((4000, D), lambda i: (i, 0)),
    out_shape=jax.ShapeDtypeStruct((NE, D), jnp.float32),
)


def _fin_body(a_ref, h_ref, d_ref, r_ref, o_ref):
    dd = d_ref[...] * d_ref[...]
    o_ref[...] = a_ref[...] + jnp.maximum(h_ref[...] + r_ref[...], 0.0) * dd


_fin_call = pl.pallas_call(
    _fin_body,
    grid=(5,),
    in_specs=[
        pl.BlockSpec((N // 5, D), lambda i: (i, 0)),
        pl.BlockSpec((N // 5, D), lambda i: (i, 0)),
        pl.BlockSpec((N // 5, 1), lambda i: (i, 0)),
        pl.BlockSpec((1, D), lambda i: (0, 0)),
    ],
    out_specs=pl.BlockSpec((N // 5, D), lambda i: (i, 0)),
    out_shape=jax.ShapeDtypeStruct((N, D), jnp.float32),
)


@jax.jit
def kernel(x, edge_index, edge_attr, W_lin, b_lin, W_edge, b_edge, root_emb):
    ei = edge_index.astype(jnp.int32)
    row = ei[0]
    col = ei[1]
    attr_pad = jnp.pad(edge_attr, ((0, 0), (0, EDP - ED)))
    w_e = jnp.pad(W_edge.T, ((0, EDP - ED), (0, 0)))
    h = _h_call(x, W_lin, b_lin.reshape(1, D))
    e = _e_call(attr_pad, w_e, b_edge.reshape(1, D))
    dinv = _sc_count(row)
    aggr = _sc_main(row, col, h, e, dinv)
    out = _fin_call(aggr.reshape(NPAD, D), h, dinv.reshape(NPAD, 1),
                    root_emb.reshape(1, D))
    return out


# speedup vs baseline: 1.5803x; 1.0130x over previous
import functools

import jax
import jax.numpy as jnp
from jax import lax
from jax.experimental import pallas as pl
from jax.experimental.pallas import tpu as pltpu
from jax.experimental.pallas import tpu_sc as plsc

N = 10000
NPAD = 10240
NE = 160000
D = 256
ED = 7
EDP = 8

NC, NS, L = 2, 16, 16
NW = NC * NS
NV = D // L
NPASS = 2
RPP = NPAD // NW // NPASS

BLK = 800
VPB = BLK // L
NBLK = NE // BLK

K = 64
NB = 2
THRESH = 1280
WLSZ = THRESH + BLK + 80


def _rsqrt_vec(d):
    i = lax.bitcast_convert_type(d, jnp.int32)
    i = jnp.int32(0x5F3759DF) - (i >> 1)
    y = lax.bitcast_convert_type(i, jnp.float32)
    for _ in range(3):
        y = y * (1.5 - 0.5 * d * y * y)
    return y


def _cnt_body(row_hbm, dinv_hbm, idx_blk, ones_blk, dinv, cnt_sh):
    c = lax.axis_index("c")
    s = lax.axis_index("s")
    zf = jnp.zeros((L,), jnp.float32)

    def _fill_ones(i, carry):
        ones_blk[pl.ds(i * L, L)] = zf + 1.0
        return carry
    lax.fori_loop(0, BLK // L, _fill_ones, 0)

    def _zero_dinv(i, carry):
        dinv[pl.ds(i * L, L)] = zf
        return carry
    lax.fori_loop(0, NPAD // L, _zero_dinv, 0)

    @pl.when((s == 0) & (c == 0))
    def _():
        pltpu.sync_copy(dinv, cnt_sh)
    plsc.subcore_barrier()

    @pl.when(c == 0)
    def _():
        def _count_blk(b, carry):
            idx = b * NS + s

            @pl.when(idx < NBLK)
            def _():
                pltpu.sync_copy(row_hbm.at[pl.ds(idx * BLK, BLK)], idx_blk)
                pltpu.sync_copy(ones_blk, cnt_sh.at[idx_blk], add=True)
            return carry
        lax.fori_loop(0, (NBLK + NS - 1) // NS, _count_blk, 0)
    plsc.subcore_barrier()

    @pl.when((s == 0) & (c == 0))
    def _():
        pltpu.sync_copy(cnt_sh, dinv)

        def _mk_dinv(i, carry):
            d = dinv[pl.ds(i * L, L)] + 1.0
            dinv[pl.ds(i * L, L)] = _rsqrt_vec(d)
            return carry
        lax.fori_loop(0, NPAD // L, _mk_dinv, 0)
        pltpu.sync_copy(dinv, dinv_hbm)


_sc_count = pl.kernel(
    _cnt_body,
    out_type=jax.ShapeDtypeStruct((NPAD,), jnp.float32),
    mesh=plsc.VectorSubcoreMesh(core_axis_name="c", subcore_axis_name="s"),
    compiler_params=pltpu.CompilerParams(needs_layout_passes=False),
    scratch_types=[
        pltpu.VMEM((BLK,), jnp.int32),
        pltpu.VMEM((BLK,), jnp.float32),
        pltpu.VMEM((NPAD,), jnp.float32),
        pltpu.VMEM_SHARED((NPAD,), jnp.float32),
    ],
)


def _sc_body(row_hbm, col_hbm, h_hbm, e_hbm, dinv_hbm, out_hbm,
             rb0, rb1, cb0, cb1, dinv, wl_row, wl_eid, wl_col,
             wlb_row, wlb_eid, wlb_col,
             nrm_buf, acc, h_b0, h_b1, e_b0, e_b1,
             sh0, sh1, se0, se1, sr0, sr1, sc0, sc1):
    c = lax.axis_index("c")
    s = lax.axis_index("s")
    wid = s * NC + c
    zf = jnp.zeros((L,), jnp.float32)
    zi = jnp.zeros((L,), jnp.int32)
    iota = lax.iota(jnp.int32, L)
    h_bufs = (h_b0, h_b1)
    e_bufs = (e_b0, e_b1)
    sems_h = (sh0, sh1)
    sems_e = (se0, se1)
    rbufs = (rb0, rb1)
    cbufs = (cb0, cb1)
    sems_r = (sr0, sr1)
    sems_c = (sc0, sc1)

    pltpu.sync_copy(dinv_hbm, dinv.at[pl.ds(0, NPAD)])

    def _start(k, b):
        pltpu.make_async_copy(
            h_hbm.at[wl_row.at[pl.ds(k * K, K)]], h_bufs[b], sems_h[b]).start()
        pltpu.make_async_copy(
            e_hbm.at[wl_eid.at[pl.ds(k * K, K)]], e_bufs[b], sems_e[b]).start()

    def _make_pass(p):
        vbase = wid * NPASS + p
        rbase = vbase * RPP

        def _zero_acc(j, carry):
            acc[pl.ds(j * L, L)] = zf
            return carry
        lax.fori_loop(0, RPP * D // L, _zero_acc, 0)

        def _finish(k, b, lim):
            pltpu.make_async_copy(
                h_hbm.at[wl_row.at[pl.ds(k * K, K)]], h_bufs[b],
                sems_h[b]).wait()
            pltpu.make_async_copy(
                e_hbm.at[wl_eid.at[pl.ds(k * K, K)]], e_bufs[b],
                sems_e[b]).wait()
            nv = lim - k * K
            for t in range(K // L):
                rv = wl_row[pl.ds(k * K + t * L, L)]
                cv = wl_col[pl.ds(k * K + t * L, L)]
                nrm = (plsc.load_gather(dinv, [rv]) *
                       plsc.load_gather(dinv, [cv]))
                valid = (iota + t * L) < nv
                nrm_buf[pl.ds(t * L, L)] = jnp.where(valid, nrm, 0.0)
            h_buf = h_bufs[b]
            e_buf = e_bufs[b]

            def _edge(j, carry):
                nrm = nrm_buf[pl.ds(j, L)][0]
                cl = wl_col[pl.ds(k * K + j, L)][0] - rbase
                base = cl << 8
                for g in range(NV // 4):
                    hv = [h_buf[j, pl.ds((g * 4 + i) * L, L)]
                          for i in range(4)]
                    ev = [e_buf[j, pl.ds((g * 4 + i) * L, L)]
                          for i in range(4)]
                    sv = [hv[i] + ev[i] for i in range(4)]
                    mv = [jnp.maximum(sv[i], 0.0) * nrm for i in range(4)]
                    for i in range(4):
                        plsc.addupdate(
                            acc.at[pl.ds(base + (g * 4 + i) * L, L)], mv[i])
                return carry
            lax.fori_loop(0, K, _edge, 0, unroll=4)

        def _drain(nch, lim):
            @pl.when(0 < nch)
            def _():
                _start(0, 0)
            ngroups = (nch + 1) >> 1

            def _grp(g, carry):
                k0 = g * NB
                for b in range(NB):
                    k = k0 + b
                    nxt = k + NB - 1

                    @pl.when(nxt < nch)
                    def _(nxt=nxt, b=b):
                        _start(nxt, (b + NB - 1) % NB)

                    @pl.when(k < nch)
                    def _(k=k, b=b):
                        _finish(k, b, lim)
                return carry
            lax.fori_loop(0, ngroups, _grp, 0)

        def _scan_pair(bufs, bbase, g, offs):
            offa, offb = offs
            va = 2 * g
            vb = 2 * g + 1
            ra = bufs[0][pl.ds(va * L, L)]
            ca = bufs[1][pl.ds(va * L, L)]
            rb = bufs[0][pl.ds(vb * L, L)]
            cb = bufs[1][pl.ds(vb * L, L)]
            ma = (((ca >> 5) * 6554) >> 15) == vbase
            mb = (((cb >> 5) * 6554) >> 15) == vbase
            ea = (bbase + va * L) + iota
            eb = (bbase + vb * L) + iota
            plsc.store_compressed(wl_row.at[pl.ds(offa, L)], ra, mask=ma)
            plsc.store_compressed(wlb_row.at[pl.ds(offb, L)], rb, mask=mb)
            plsc.store_compressed(wl_eid.at[pl.ds(offa, L)], ea, mask=ma)
            plsc.store_compressed(wlb_eid.at[pl.ds(offb, L)], eb, mask=mb)
            plsc.store_compressed(wl_col.at[pl.ds(offa, L)], ca, mask=ma)
            plsc.store_compressed(wlb_col.at[pl.ds(offb, L)], cb, mask=mb)
            pa = plsc.all_reduce_population_count(ma)[0]
            pb = plsc.all_reduce_population_count(mb)[0]
            return offa + pa, offb + pb

        def _start_blk(b, buf):
            pltpu.make_async_copy(
                row_hbm.at[pl.ds(b * BLK, BLK)], rbufs[buf],
                sems_r[buf]).start()
            pltpu.make_async_copy(
                col_hbm.at[pl.ds(b * BLK, BLK)], cbufs[buf],
                sems_c[buf]).start()

        def _scan_blk(buf, b, off):
            pltpu.make_async_copy(
                row_hbm.at[pl.ds(b * BLK, BLK)], rbufs[buf],
                sems_r[buf]).wait()
            pltpu.make_async_copy(
                col_hbm.at[pl.ds(b * BLK, BLK)], cbufs[buf],
                sems_c[buf]).wait()
            off, offb = lax.fori_loop(
                0, VPB // 2,
                functools.partial(_scan_pair, (rbufs[buf], cbufs[buf]),
                                  b * BLK),
                (off, jnp.int32(0)))

            def _merge(t, carry):
                mr = wlb_row[pl.ds(t * L, L)]
                me = wlb_eid[pl.ds(t * L, L)]
                mc = wlb_col[pl.ds(t * L, L)]
                wl_row[pl.ds(off + t * L, L)] = mr
                wl_eid[pl.ds(off + t * L, L)] = me
                wl_col[pl.ds(off + t * L, L)] = mc
                return carry
            lax.fori_loop(0, (offb + (L - 1)) >> 4, _merge, 0)
            return off + offb

        def _pad_tail(off):
            for t in range(K // L):
                wl_row[pl.ds(off + t * L, L)] = zi
                wl_eid[pl.ds(off + t * L, L)] = zi
                wl_col[pl.ds(off + t * L, L)] = zi + rbase
            return off

        _start_blk(0, 0)

        def _body(g, off):
            for sub in range(NB):
                b = g * NB + sub

                def _real(off, b=b, sub=sub):
                    nxt = b + 1

                    @pl.when(nxt < NBLK)
                    def _():
                        _start_blk(nxt, (sub + 1) % NB)

                    is_last = b == NBLK
                    off = lax.cond(
                        is_last, _pad_tail,
                        functools.partial(_scan_blk, sub, b), off)
                    do = (off >= THRESH) | is_last
                    nch = lax.select(is_last, (off + (K - 1)) >> 6, off >> 6)
                    lim = nch << 6

                    def _do_drain(off):
                        _drain(nch, jnp.minimum(lim, off))
                        base = (off >> 6) << 6
                        for t in range(K // L):
                            lr = wl_row[pl.ds(base + t * L, L)]
                            le = wl_eid[pl.ds(base + t * L, L)]
                            lc = wl_col[pl.ds(base + t * L, L)]
                            wl_row[pl.ds(t * L, L)] = lr
                            wl_eid[pl.ds(t * L, L)] = le
                            wl_col[pl.ds(t * L, L)] = lc
                        return off - base

                    return lax.cond(do, _do_drain, lambda o: o, off)

                off = lax.cond(b <= NBLK, _real, lambda o: o, off)
            return off
        lax.fori_loop(0, (NBLK + 2 + NB - 1) // NB, _body, jnp.int32(0))

        pltpu.sync_copy(acc, out_hbm.at[pl.ds(rbase * D, RPP * D)])

    for p in range(NPASS):
        _make_pass(p)


_sc_main = pl.kernel(
    _sc_body,
    out_type=jax.ShapeDtypeStruct((NPAD * D,), jnp.float32),
    mesh=plsc.VectorSubcoreMesh(core_axis_name="c", subcore_axis_name="s"),
    compiler_params=pltpu.CompilerParams(needs_layout_passes=False),
    scratch_types=[
        pltpu.VMEM((BLK,), jnp.int32),
        pltpu.VMEM((BLK,), jnp.int32),
        pltpu.VMEM((BLK,), jnp.int32),
        pltpu.VMEM((BLK,), jnp.int32),
        pltpu.VMEM((NPAD,), jnp.float32),
        pltpu.VMEM((WLSZ,), jnp.int32),
        pltpu.VMEM((WLSZ,), jnp.int32),
        pltpu.VMEM((WLSZ,), jnp.int32),
        pltpu.VMEM((BLK // 2 + L,), jnp.int32),
        pltpu.VMEM((BLK // 2 + L,), jnp.int32),
        pltpu.VMEM((BLK // 2 + L,), jnp.int32),
        pltpu.VMEM((K + L,), jnp.float32),
        pltpu.VMEM((RPP * D,), jnp.float32),
        pltpu.VMEM((K, D), jnp.float32),
        pltpu.VMEM((K, D), jnp.float32),
        pltpu.VMEM((K, D), jnp.float32),
        pltpu.VMEM((K, D), jnp.float32),
        pltpu.SemaphoreType.DMA,
        pltpu.SemaphoreType.DMA,
        pltpu.SemaphoreType.DMA,
        pltpu.SemaphoreType.DMA,
        pltpu.SemaphoreType.DMA,
        pltpu.SemaphoreType.DMA,
        pltpu.SemaphoreType.DMA,
        pltpu.SemaphoreType.DMA,
    ],
)


def _h_body(x_ref, w_ref, b_ref, o_ref):
    o_ref[...] = lax.dot_general(
        x_ref[...], w_ref[...], (((1,), (1,)), ((), ())),
        preferred_element_type=jnp.float32) + b_ref[...]


_h_call = pl.pallas_call(
    _h_body,
    grid=(5,),
    in_specs=[
        pl.BlockSpec((N // 5, D), lambda i: (i, 0)),
        pl.BlockSpec((D, D), lambda i: (0, 0)),
        pl.BlockSpec((1, D), lambda i: (0, 0)),
    ],
    out_specs=pl.BlockSpec((N // 5, D), lambda i: (i, 0)),
    out_shape=jax.ShapeDtypeStruct((N, D), jnp.float32),
)


def _e_body(a_ref, w_ref, b_ref, o_ref):
    o_ref[...] = lax.dot_general(
        a_ref[...], w_ref[...], (((1,), (0,)), ((), ())),
        preferred_element_type=jnp.float32) + b_ref[...]


_e_call = pl.pallas_call(
    _e_body,
    grid=(40,),
    in_specs=[
        pl.BlockSpec((4000, EDP), lambda i: (i, 0)),
        pl.BlockSpec((EDP, D), lambda i: (0, 0)),
        pl.BlockSpec((1, D), lambda i: (0, 0)),
    ],
    out_specs=pl.BlockSpec((4000, D), lambda i: (i, 0)),
    out_shape=jax.ShapeDtypeStruct((NE, D), jnp.float32),
)


def _fin_body(a_ref, h_ref, d_ref, r_ref, o_ref):
    dd = d_ref[...] * d_ref[...]
    o_ref[...] = a_ref[...] + jnp.maximum(h_ref[...] + r_ref[...], 0.0) * dd


_fin_call = pl.pallas_call(
    _fin_body,
    grid=(5,),
    in_specs=[
        pl.BlockSpec((N // 5, D), lambda i: (i, 0)),
        pl.BlockSpec((N // 5, D), lambda i: (i, 0)),
        pl.BlockSpec((N // 5, 1), lambda i: (i, 0)),
        pl.BlockSpec((1, D), lambda i: (0, 0)),
    ],
    out_specs=pl.BlockSpec((N // 5, D), lambda i: (i, 0)),
    out_shape=jax.ShapeDtypeStruct((N, D), jnp.float32),
)


@jax.jit
def kernel(x, edge_index, edge_attr, W_lin, b_lin, W_edge, b_edge, root_emb):
    ei = edge_index.astype(jnp.int32)
    row = ei[0]
    col = ei[1]
    attr_pad = jnp.pad(edge_attr, ((0, 0), (0, EDP - ED)))
    w_e = jnp.pad(W_edge.T, ((0, EDP - ED), (0, 0)))
    h = _h_call(x, W_lin, b_lin.reshape(1, D))
    e = _e_call(attr_pad, w_e, b_edge.reshape(1, D))
    dinv = _sc_count(row)
    aggr = _sc_main(row, col, h, e, dinv)
    out = _fin_call(aggr.reshape(NPAD, D), h, dinv.reshape(NPAD, 1),
                    root_emb.reshape(1, D))
    return out
